# Initial kernel scaffold; baseline (speedup 1.0000x reference)
import jax
import jax.numpy as jnp
from jax.experimental import pallas as pl


def kernel(x, edge_index, w_embed, w_classify):
    raise NotImplementedError("write your pallas kernel here")


# speedup vs baseline: 2.6963x; 2.6963x over previous
import functools

import jax
import jax.numpy as jnp
from jax import lax
from jax.experimental import pallas as pl
from jax.experimental.pallas import tpu as pltpu
from jax.experimental.pallas import tpu_sc as plsc

_NC = 2
_NS = 16
_NWORKERS = _NC * _NS


def _round_up(x, m):
    return (x + m - 1) // m * m


def _sc_build_adj(src, dst, n_pad, w_words, e_pad, chunk):
    n_chunks = e_pad // chunk
    groups = chunk // 16

    mesh = plsc.VectorSubcoreMesh(core_axis_name="c", subcore_axis_name="s",
                                  num_cores=_NC, num_subcores=_NS)

    @functools.partial(
        pl.kernel,
        mesh=mesh,
        out_type=[
            jax.ShapeDtypeStruct((n_pad * w_words,), jnp.int32),
            jax.ShapeDtypeStruct((n_pad,), jnp.int32),
        ],
        scratch_types=[
            pltpu.VMEM((chunk,), jnp.int32),
            pltpu.VMEM((chunk,), jnp.int32),
            pltpu.VMEM((chunk + 32,), jnp.int32),
            pltpu.VMEM((chunk + 32,), jnp.int32),
            pltpu.VMEM(((w_words + 1) * w_words,), jnp.int32),
            pltpu.VMEM((w_words + 16,), jnp.int32),
        ],
        compiler_params=pltpu.CompilerParams(needs_layout_passes=False),
    )
    def build(src_hbm, dst_hbm, f_hbm, self_hbm, s_v, d_v, sel_s, sel_d,
              acc, selfc):
        wid = lax.axis_index("s") * _NC + lax.axis_index("c")
        base = wid * w_words
        zero16 = jnp.zeros((16,), jnp.int32)

        def zero_acc(i, _):
            acc[pl.ds(i * 16, 16)] = zero16
            return 0

        lax.fori_loop(0, (w_words * w_words) // 16, zero_acc, 0)

        def zero_selfc(i, _):
            selfc[pl.ds(i * 16, 16)] = zero16
            return 0

        lax.fori_loop(0, w_words // 16 + 1, zero_selfc, 0)

        def do_chunk(c, _):
            pltpu.sync_copy(src_hbm.at[pl.ds(c * chunk, chunk)], s_v)
            pltpu.sync_copy(dst_hbm.at[pl.ds(c * chunk, chunk)], d_v)

            def compact(g, cnt):
                s16 = s_v[pl.ds(g * 16, 16)]
                d16 = d_v[pl.ds(g * 16, 16)]
                m = jnp.logical_and(s16 >= base, s16 < base + w_words)
                mi = jnp.where(m, 1, 0)
                cs = plsc.cumsum(mi)
                pos = cnt + cs - mi
                plsc.store_scatter(sel_s, [pos], s16 - base, mask=m)
                plsc.store_scatter(sel_d, [pos], d16, mask=m)
                return cnt + cs[15]

            cnt = lax.fori_loop(0, groups, compact, 0)

            lane0 = lax.iota(jnp.int32, 16) == 0

            def scatter_bit(p, _):
                s_loc = sel_s[pl.ds(p, 16)][0]
                dd = sel_d[pl.ds(p, 16)][0]
                wd = lax.rem(dd, w_words)
                bit = lax.div(dd, w_words)
                addr = s_loc * w_words + wd
                win = acc[pl.ds(addr, 16)]
                acc[pl.ds(addr, 16)] = jnp.where(
                    lane0, win | (jnp.int32(1) << bit), win)

                @pl.when(s_loc + base == dd)
                def _():
                    cwin = selfc[pl.ds(s_loc, 16)]
                    selfc[pl.ds(s_loc, 16)] = jnp.where(lane0, cwin + 1, cwin)

                return 0

            lax.fori_loop(0, cnt, scatter_bit, 0)
            return 0

        lax.fori_loop(0, n_chunks, do_chunk, 0)

        pltpu.sync_copy(acc.at[pl.ds(0, w_words * w_words)],
                        f_hbm.at[pl.ds(base * w_words, w_words * w_words)])
        pltpu.sync_copy(selfc.at[pl.ds(0, w_words)],
                        self_hbm.at[pl.ds(base, w_words)])

    return build(src, dst)


def _sc_two_hop(src, dst, f2d, n_pad, w_words, e_pad, chunk):
    n_chunks = e_pad // chunk
    groups = chunk // 16
    g_pitch = f2d.shape[1]

    mesh = plsc.VectorSubcoreMesh(core_axis_name="c", subcore_axis_name="s",
                                  num_cores=_NC, num_subcores=_NS)

    @functools.partial(
        pl.kernel,
        mesh=mesh,
        out_type=jax.ShapeDtypeStruct((n_pad * w_words,), jnp.int32),
        scratch_types=[
            pltpu.VMEM((chunk,), jnp.int32),
            pltpu.VMEM((chunk,), jnp.int32),
            pltpu.VMEM((chunk + 32,), jnp.int32),
            pltpu.VMEM((chunk + 32,), jnp.int32),
            pltpu.VMEM(((w_words + 1) * w_words,), jnp.int32),
            pltpu.VMEM((16, g_pitch), jnp.int32),
            pltpu.SemaphoreType.DMA,
        ],
        compiler_params=pltpu.CompilerParams(needs_layout_passes=False),
    )
    def twohop(src_hbm, dst_hbm, f_hbm, two_hbm, s_v, d_v, sel_s, sel_d,
               acc, rows, sem):
        wid = lax.axis_index("s") * _NC + lax.axis_index("c")
        base = wid * w_words
        zero16 = jnp.zeros((16,), jnp.int32)

        def zero_acc(i, _):
            acc[pl.ds(i * 16, 16)] = zero16
            return 0

        lax.fori_loop(0, (w_words * w_words) // 16, zero_acc, 0)

        def do_chunk(c, _):
            pltpu.sync_copy(src_hbm.at[pl.ds(c * chunk, chunk)], s_v)
            pltpu.sync_copy(dst_hbm.at[pl.ds(c * chunk, chunk)], d_v)

            def compact(g, cnt):
                s16 = s_v[pl.ds(g * 16, 16)]
                d16 = d_v[pl.ds(g * 16, 16)]
                m = jnp.logical_and(s16 >= base, s16 < base + w_words)
                mi = jnp.where(m, 1, 0)
                cs = plsc.cumsum(mi)
                pos = cnt + cs - mi
                plsc.store_scatter(sel_s, [pos], s16 - base, mask=m)
                plsc.store_scatter(sel_d, [pos], d16, mask=m)
                return cnt + cs[15]

            cnt = lax.fori_loop(0, groups, compact, 0)

            sel_s[pl.ds(cnt, 16)] = jnp.full((16,), w_words, jnp.int32)
            sel_d[pl.ds(cnt, 16)] = zero16

            def do_group(g, _):
                pltpu.async_copy(f_hbm.at[sel_d.at[pl.ds(g * 16, 16)]], rows,
                                 sem).wait()
                svec = sel_s[pl.ds(g * 16, 16)]
                for l in range(16):
                    s_loc = svec[l]
                    for cc in range(w_words // 16):
                        off = s_loc * w_words + cc * 16
                        acc[pl.ds(off, 16)] = (acc[pl.ds(off, 16)]
                                               | rows[l, pl.ds(cc * 16, 16)])
                return 0

            lax.fori_loop(0, (cnt + 15) // 16, do_group, 0)
            return 0

        lax.fori_loop(0, n_chunks, do_chunk, 0)

        pltpu.sync_copy(acc.at[pl.ds(0, w_words * w_words)],
                        two_hbm.at[pl.ds(base * w_words, w_words * w_words)])

    return twohop(src, dst, f2d)


def _unpack_chunks(f_blk, two_blk, self2, i0, bi, w_words):
    a2w = two_blk & ~f_blk
    row_ids = i0 + lax.broadcasted_iota(jnp.int32, (bi, w_words), 0)
    col_ids = lax.broadcasted_iota(jnp.int32, (bi, w_words), 1)
    one = jnp.int32(1)
    for j in range(32):
        diag = row_ids == (j * w_words + col_ids)
        a1 = ((f_blk >> j) & one).astype(jnp.float32)
        a2 = ((a2w >> j) & one).astype(jnp.float32)
        a1 = jnp.where(diag, self2, a1)
        a2 = jnp.where(diag, 0.0, a2)
        yield j, a1, a2


def _tc_degrees(f2d, two2d, selfcnt2d, n_pad, w_words, bi):
    nb = n_pad // bi

    def body(f_ref, t_ref, sc_ref, d1_ref, d2_ref):
        i0 = pl.program_id(0) * bi
        f_blk = f_ref[...]
        t_blk = t_ref[...]
        self2 = (sc_ref[...] >= 2).astype(jnp.float32)
        d1 = jnp.zeros((bi, 1), jnp.float32)
        d2 = jnp.zeros((bi, 1), jnp.float32)
        for j, a1, a2 in _unpack_chunks(f_blk, t_blk, self2, i0, bi, w_words):
            d1 = d1 + jnp.sum(a1, axis=1, keepdims=True)
            d2 = d2 + jnp.sum(a2, axis=1, keepdims=True)
        d1_ref[...] = jnp.where(d1 > 0, lax.rsqrt(d1), 0.0)
        d2_ref[...] = jnp.where(d2 > 0, lax.rsqrt(d2), 0.0)

    return pl.pallas_call(
        body,
        grid=(nb,),
        in_specs=[
            pl.BlockSpec((bi, w_words), lambda i: (i, 0)),
            pl.BlockSpec((bi, w_words), lambda i: (i, 0)),
            pl.BlockSpec((bi, 1), lambda i: (i, 0)),
        ],
        out_specs=[
            pl.BlockSpec((bi, 1), lambda i: (i, 0)),
            pl.BlockSpec((bi, 1), lambda i: (i, 0)),
        ],
        out_shape=[
            jax.ShapeDtypeStruct((n_pad, 1), jnp.float32),
            jax.ShapeDtypeStruct((n_pad, 1), jnp.float32),
        ],
        compiler_params=pltpu.CompilerParams(
            dimension_semantics=("parallel",)),
    )(f2d, two2d, selfcnt2d)


def _tc_embed(x_pad, w_embed, d1inv, d2inv, n_pad, bi):
    nb = n_pad // bi
    feat = x_pad.shape[1]
    hid = w_embed.shape[1]

    def body(x_ref, w_ref, d1_ref, d2_ref, r_ref, rv1_ref, rv2_ref):
        r = jnp.maximum(
            jnp.dot(x_ref[...], w_ref[...],
                    preferred_element_type=jnp.float32), 0.0)
        r_ref[...] = r
        rv1_ref[...] = r * d1_ref[...]
        rv2_ref[...] = r * d2_ref[...]

    return pl.pallas_call(
        body,
        grid=(nb,),
        in_specs=[
            pl.BlockSpec((bi, feat), lambda i: (i, 0)),
            pl.BlockSpec((feat, hid), lambda i: (0, 0)),
            pl.BlockSpec((bi, 1), lambda i: (i, 0)),
            pl.BlockSpec((bi, 1), lambda i: (i, 0)),
        ],
        out_specs=[
            pl.BlockSpec((bi, hid), lambda i: (i, 0)),
            pl.BlockSpec((bi, hid), lambda i: (i, 0)),
            pl.BlockSpec((bi, hid), lambda i: (i, 0)),
        ],
        out_shape=[
            jax.ShapeDtypeStruct((n_pad, hid), jnp.float32),
            jax.ShapeDtypeStruct((n_pad, hid), jnp.float32),
            jax.ShapeDtypeStruct((n_pad, hid), jnp.float32),
        ],
        compiler_params=pltpu.CompilerParams(
            dimension_semantics=("parallel",)),
    )(x_pad, w_embed, d1inv, d2inv)


def _tc_hop1(f2d, two2d, selfcnt2d, d1inv, d2inv, rv1, rv2, n_pad, w_words,
             bi):
    nb = n_pad // bi
    hid = rv1.shape[1]

    def body(f_ref, t_ref, sc_ref, d1_ref, d2_ref, v1_ref, v2_ref,
             y_ref, yv1_ref, yv2_ref):
        i0 = pl.program_id(0) * bi
        f_blk = f_ref[...]
        t_blk = t_ref[...]
        self2 = (sc_ref[...] >= 2).astype(jnp.float32)
        acc1 = jnp.zeros((bi, hid), jnp.float32)
        acc2 = jnp.zeros((bi, hid), jnp.float32)
        for j, a1, a2 in _unpack_chunks(f_blk, t_blk, self2, i0, bi, w_words):
            v1 = v1_ref[pl.ds(j * w_words, w_words), :]
            v2 = v2_ref[pl.ds(j * w_words, w_words), :]
            acc1 = acc1 + jnp.dot(a1, v1, preferred_element_type=jnp.float32)
            acc2 = acc2 + jnp.dot(a2, v2, preferred_element_type=jnp.float32)
        d1 = d1_ref[...]
        d2 = d2_ref[...]
        y = jnp.concatenate([acc1 * d1, acc2 * d2], axis=1)
        y_ref[...] = y
        yv1_ref[...] = y * d1
        yv2_ref[...] = y * d2

    return pl.pallas_call(
        body,
        grid=(nb,),
        in_specs=[
            pl.BlockSpec((bi, w_words), lambda i: (i, 0)),
            pl.BlockSpec((bi, w_words), lambda i: (i, 0)),
            pl.BlockSpec((bi, 1), lambda i: (i, 0)),
            pl.BlockSpec((bi, 1), lambda i: (i, 0)),
            pl.BlockSpec((bi, 1), lambda i: (i, 0)),
            pl.BlockSpec((n_pad, hid), lambda i: (0, 0)),
            pl.BlockSpec((n_pad, hid), lambda i: (0, 0)),
        ],
        out_specs=[
            pl.BlockSpec((bi, 2 * hid), lambda i: (i, 0)),
            pl.BlockSpec((bi, 2 * hid), lambda i: (i, 0)),
            pl.BlockSpec((bi, 2 * hid), lambda i: (i, 0)),
        ],
        out_shape=[
            jax.ShapeDtypeStruct((n_pad, 2 * hid), jnp.float32),
            jax.ShapeDtypeStruct((n_pad, 2 * hid), jnp.float32),
            jax.ShapeDtypeStruct((n_pad, 2 * hid), jnp.float32),
        ],
        compiler_params=pltpu.CompilerParams(
            dimension_semantics=("parallel",)),
    )(f2d, two2d, selfcnt2d, d1inv, d2inv, rv1, rv2)


def _tc_hop2_classify(f2d, two2d, selfcnt2d, d1inv, d2inv, yv1, yv2, r, y,
                      w_classify, n_pad, w_words, bi):
    nb = n_pad // bi
    c2 = yv1.shape[1]
    hid = r.shape[1]
    cls = w_classify.shape[1]

    def body(f_ref, t_ref, sc_ref, d1_ref, d2_ref, v1_ref, v2_ref, r_ref,
             y_ref, wc_ref, out_ref):
        i0 = pl.program_id(0) * bi
        f_blk = f_ref[...]
        t_blk = t_ref[...]
        self2 = (sc_ref[...] >= 2).astype(jnp.float32)
        acc1 = jnp.zeros((bi, c2), jnp.float32)
        acc2 = jnp.zeros((bi, c2), jnp.float32)
        for j, a1, a2 in _unpack_chunks(f_blk, t_blk, self2, i0, bi, w_words):
            v1 = v1_ref[pl.ds(j * w_words, w_words), :]
            v2 = v2_ref[pl.ds(j * w_words, w_words), :]
            acc1 = acc1 + jnp.dot(a1, v1, preferred_element_type=jnp.float32)
            acc2 = acc2 + jnp.dot(a2, v2, preferred_element_type=jnp.float32)
        z1 = acc1 * d1_ref[...]
        z2 = acc2 * d2_ref[...]
        out = jnp.dot(r_ref[...], wc_ref[pl.ds(0, hid), :],
                      preferred_element_type=jnp.float32)
        out = out + jnp.dot(y_ref[...], wc_ref[pl.ds(hid, c2), :],
                            preferred_element_type=jnp.float32)
        out = out + jnp.dot(z1, wc_ref[pl.ds(hid + c2, c2), :],
                            preferred_element_type=jnp.float32)
        out = out + jnp.dot(z2, wc_ref[pl.ds(hid + 2 * c2, c2), :],
                            preferred_element_type=jnp.float32)
        out_ref[...] = out

    return pl.pallas_call(
        body,
        grid=(nb,),
        in_specs=[
            pl.BlockSpec((bi, w_words), lambda i: (i, 0)),
            pl.BlockSpec((bi, w_words), lambda i: (i, 0)),
            pl.BlockSpec((bi, 1), lambda i: (i, 0)),
            pl.BlockSpec((bi, 1), lambda i: (i, 0)),
            pl.BlockSpec((bi, 1), lambda i: (i, 0)),
            pl.BlockSpec((n_pad, c2), lambda i: (0, 0)),
            pl.BlockSpec((n_pad, c2), lambda i: (0, 0)),
            pl.BlockSpec((bi, hid), lambda i: (i, 0)),
            pl.BlockSpec((bi, c2), lambda i: (i, 0)),
            pl.BlockSpec(w_classify.shape, lambda i: (0, 0)),
        ],
        out_specs=pl.BlockSpec((bi, cls), lambda i: (i, 0)),
        out_shape=jax.ShapeDtypeStruct((n_pad, cls), jnp.float32),
        compiler_params=pltpu.CompilerParams(
            dimension_semantics=("parallel",)),
    )(f2d, two2d, selfcnt2d, d1inv, d2inv, yv1, yv2, r, y, w_classify)


def _tc_pipeline(f2d, two2d, selfcnt2d, x, w_embed, w_classify, n, n_pad,
                 w_words, bi):
    d1inv, d2inv = _tc_degrees(f2d, two2d, selfcnt2d, n_pad, w_words, bi)

    x_pad = jnp.pad(x, ((0, n_pad - n), (0, 0)))
    r, rv1, rv2 = _tc_embed(x_pad, w_embed, d1inv, d2inv, n_pad, bi)

    y, yv1, yv2 = _tc_hop1(f2d, two2d, selfcnt2d, d1inv, d2inv, rv1, rv2,
                           n_pad, w_words, bi)

    out = _tc_hop2_classify(f2d, two2d, selfcnt2d, d1inv, d2inv, yv1, yv2,
                            r, y, w_classify, n_pad, w_words, bi)
    return out[:n]


def kernel(x, edge_index, w_embed, w_classify):
    n = x.shape[0]
    e = edge_index.shape[1]

    w_words = _round_up(-(-n // 32), 64)
    n_pad = 32 * w_words
    bi = 256

    chunk = 1600
    e_pad = _round_up(e, chunk)

    src = edge_index[0]
    dst = edge_index[1]
    if e_pad != e:
        src = jnp.concatenate(
            [src, jnp.full((e_pad - e,), jnp.int32(2 ** 30))])
        dst = jnp.concatenate([dst, jnp.zeros((e_pad - e,), jnp.int32)])

    f_flat, selfcnt = _sc_build_adj(src, dst, n_pad, w_words, e_pad, chunk)
    f2d = f_flat.reshape(n_pad, w_words)
    g_pitch = _round_up(w_words, 128)
    f2d_g = jnp.pad(f2d, ((0, 0), (0, g_pitch - w_words)))
    two_flat = _sc_two_hop(src, dst, f2d_g, n_pad, w_words, e_pad, chunk)
    two2d = two_flat.reshape(n_pad, w_words)
    selfcnt2d = selfcnt.reshape(n_pad, 1)

    return _tc_pipeline(f2d, two2d, selfcnt2d, x, w_embed, w_classify, n,
                        n_pad, w_words, bi)


# speedup vs baseline: 2.7917x; 1.0354x over previous
import functools

import jax
import jax.numpy as jnp
from jax import lax
from jax.experimental import pallas as pl
from jax.experimental.pallas import tpu as pltpu
from jax.experimental.pallas import tpu_sc as plsc

_NC = 2
_NS = 16
_NWORKERS = _NC * _NS


def _round_up(x, m):
    return (x + m - 1) // m * m


def _sc_build_adj(src, dst, n_pad, w_words, e_pad, chunk):
    n_chunks = e_pad // chunk
    groups = chunk // 16

    mesh = plsc.VectorSubcoreMesh(core_axis_name="c", subcore_axis_name="s",
                                  num_cores=_NC, num_subcores=_NS)

    @functools.partial(
        pl.kernel,
        mesh=mesh,
        out_type=[
            jax.ShapeDtypeStruct((n_pad * w_words,), jnp.int32),
            jax.ShapeDtypeStruct((n_pad,), jnp.int32),
        ],
        scratch_types=[
            pltpu.VMEM((chunk,), jnp.int32),
            pltpu.VMEM((chunk,), jnp.int32),
            pltpu.VMEM((chunk + 32,), jnp.int32),
            pltpu.VMEM((chunk + 32,), jnp.int32),
            pltpu.VMEM(((w_words + 1) * w_words,), jnp.int32),
            pltpu.VMEM((w_words + 16,), jnp.int32),
        ],
        compiler_params=pltpu.CompilerParams(needs_layout_passes=False),
    )
    def build(src_hbm, dst_hbm, f_hbm, self_hbm, s_v, d_v, sel_s, sel_d,
              acc, selfc):
        wid = lax.axis_index("s") * _NC + lax.axis_index("c")
        base = wid * w_words
        zero16 = jnp.zeros((16,), jnp.int32)

        def zero_acc(i, _):
            acc[pl.ds(i * 16, 16)] = zero16
            return 0

        lax.fori_loop(0, (w_words * w_words) // 16, zero_acc, 0)

        def zero_selfc(i, _):
            selfc[pl.ds(i * 16, 16)] = zero16
            return 0

        lax.fori_loop(0, w_words // 16 + 1, zero_selfc, 0)

        def do_chunk(c, _):
            pltpu.sync_copy(src_hbm.at[pl.ds(c * chunk, chunk)], s_v)
            pltpu.sync_copy(dst_hbm.at[pl.ds(c * chunk, chunk)], d_v)

            def compact(g, cnt):
                s16 = s_v[pl.ds(g * 16, 16)]
                d16 = d_v[pl.ds(g * 16, 16)]
                m = jnp.logical_and(s16 >= base, s16 < base + w_words)
                mi = jnp.where(m, 1, 0)
                cs = plsc.cumsum(mi)
                pos = cnt + cs - mi
                plsc.store_scatter(sel_s, [pos], s16 - base, mask=m)
                plsc.store_scatter(sel_d, [pos], d16, mask=m)
                return cnt + cs[15]

            cnt = lax.fori_loop(0, groups, compact, 0)

            lane0 = lax.iota(jnp.int32, 16) == 0

            def scatter_bit(p, _):
                s_loc = sel_s[pl.ds(p, 16)][0]
                dd = sel_d[pl.ds(p, 16)][0]
                wd = lax.rem(dd, w_words)
                bit = lax.div(dd, w_words)
                addr = s_loc * w_words + wd
                win = acc[pl.ds(addr, 16)]
                acc[pl.ds(addr, 16)] = jnp.where(
                    lane0, win | (jnp.int32(1) << bit), win)

                @pl.when(s_loc + base == dd)
                def _():
                    cwin = selfc[pl.ds(s_loc, 16)]
                    selfc[pl.ds(s_loc, 16)] = jnp.where(lane0, cwin + 1, cwin)

                return 0

            lax.fori_loop(0, cnt, scatter_bit, 0)
            return 0

        lax.fori_loop(0, n_chunks, do_chunk, 0)

        pltpu.sync_copy(acc.at[pl.ds(0, w_words * w_words)],
                        f_hbm.at[pl.ds(base * w_words, w_words * w_words)])
        pltpu.sync_copy(selfc.at[pl.ds(0, w_words)],
                        self_hbm.at[pl.ds(base, w_words)])

    return build(src, dst)


def _sc_two_hop(src, dst, f2d, n_pad, w_words, e_pad, chunk):
    n_chunks = e_pad // chunk
    groups = chunk // 16
    g_pitch = f2d.shape[1]

    mesh = plsc.VectorSubcoreMesh(core_axis_name="c", subcore_axis_name="s",
                                  num_cores=_NC, num_subcores=_NS)

    @functools.partial(
        pl.kernel,
        mesh=mesh,
        out_type=jax.ShapeDtypeStruct((n_pad * w_words,), jnp.int32),
        scratch_types=[
            pltpu.VMEM((chunk,), jnp.int32),
            pltpu.VMEM((chunk,), jnp.int32),
            pltpu.VMEM((chunk + 48,), jnp.int32),
            pltpu.VMEM((chunk + 48,), jnp.int32),
            pltpu.VMEM(((w_words + 1) * w_words,), jnp.int32),
            pltpu.VMEM((16, g_pitch), jnp.int32),
            pltpu.VMEM((16, g_pitch), jnp.int32),
            pltpu.VMEM((16, g_pitch), jnp.int32),
            pltpu.SemaphoreType.DMA,
            pltpu.SemaphoreType.DMA,
            pltpu.SemaphoreType.DMA,
        ],
        compiler_params=pltpu.CompilerParams(needs_layout_passes=False),
    )
    def twohop(src_hbm, dst_hbm, f_hbm, two_hbm, s_v, d_v, sel_s, sel_d,
               acc, r0, r1, r2, sm0, sm1, sm2):
        wid = lax.axis_index("s") * _NC + lax.axis_index("c")
        base = wid * w_words
        zero16 = jnp.zeros((16,), jnp.int32)
        bufs = ((r0, sm0), (r1, sm1), (r2, sm2))

        def zero_acc(i, _):
            acc[pl.ds(i * 16, 16)] = zero16
            return 0

        lax.fori_loop(0, (w_words * w_words) // 16, zero_acc, 0)

        def issue(g, b):
            rbuf, sem = bufs[b]
            pltpu.async_copy(f_hbm.at[sel_d.at[pl.ds(g * 16, 16)]], rbuf, sem)

        def do_chunk(c, _):
            pltpu.sync_copy(src_hbm.at[pl.ds(c * chunk, chunk)], s_v)
            pltpu.sync_copy(dst_hbm.at[pl.ds(c * chunk, chunk)], d_v)

            def compact(g, cnt):
                s16 = s_v[pl.ds(g * 16, 16)]
                d16 = d_v[pl.ds(g * 16, 16)]
                m = jnp.logical_and(s16 >= base, s16 < base + w_words)
                mi = jnp.where(m, 1, 0)
                cs = plsc.cumsum(mi)
                pos = cnt + cs - mi
                plsc.store_scatter(sel_s, [pos], s16 - base, mask=m)
                plsc.store_scatter(sel_d, [pos], d16, mask=m)
                return cnt + cs[15]

            cnt = lax.fori_loop(0, groups, compact, 0)

            for t in range(3):
                sel_s[pl.ds(cnt + 16 * t, 16)] = jnp.full(
                    (16,), w_words, jnp.int32)
                sel_d[pl.ds(cnt + 16 * t, 16)] = zero16

            ngroups = lax.div(cnt + 15, 16)

            @pl.when(ngroups > 0)
            def _():
                issue(jnp.int32(0), 0)

            @pl.when(ngroups > 1)
            def _():
                issue(jnp.int32(1), 1)

            def or_group(g, b):
                rbuf, sem = bufs[b]
                pltpu.make_async_copy(
                    f_hbm.at[sel_d.at[pl.ds(0, 16)]], rbuf, sem).wait()

                @pl.when(g + 2 < ngroups)
                def _():
                    issue(g + 2, (b + 2) % 3)

                svec = sel_s[pl.ds(g * 16, 16)]
                for l in range(16):
                    s_loc = svec[l]
                    for cc in range(w_words // 16):
                        off = s_loc * w_words + cc * 16
                        acc[pl.ds(off, 16)] = (acc[pl.ds(off, 16)]
                                               | rbuf[l, pl.ds(cc * 16, 16)])

            def do_trip(t, _):
                g = t * 3

                @pl.when(g < ngroups)
                def _():
                    or_group(g, 0)

                @pl.when(g + 1 < ngroups)
                def _():
                    or_group(g + 1, 1)

                @pl.when(g + 2 < ngroups)
                def _():
                    or_group(g + 2, 2)

                return 0

            lax.fori_loop(0, (ngroups + 2) // 3, do_trip, 0)
            return 0

        lax.fori_loop(0, n_chunks, do_chunk, 0)

        pltpu.sync_copy(acc.at[pl.ds(0, w_words * w_words)],
                        two_hbm.at[pl.ds(base * w_words, w_words * w_words)])

    return twohop(src, dst, f2d)


def _unpack_chunks(f_blk, two_blk, self2, i0, bi, w_words):
    a2w = two_blk & ~f_blk
    row_ids = i0 + lax.broadcasted_iota(jnp.int32, (bi, w_words), 0)
    col_ids = lax.broadcasted_iota(jnp.int32, (bi, w_words), 1)
    one = jnp.int32(1)
    for j in range(32):
        diag = row_ids == (j * w_words + col_ids)
        a1 = ((f_blk >> j) & one).astype(jnp.float32)
        a2 = ((a2w >> j) & one).astype(jnp.float32)
        a1 = jnp.where(diag, self2, a1)
        a2 = jnp.where(diag, 0.0, a2)
        yield j, a1, a2


def _tc_degrees(f2d, two2d, selfcnt2d, n_pad, w_words, bi):
    nb = n_pad // bi

    def body(f_ref, t_ref, sc_ref, d1_ref, d2_ref):
        i0 = pl.program_id(0) * bi
        f_blk = f_ref[...]
        t_blk = t_ref[...]
        self2 = (sc_ref[...] >= 2).astype(jnp.float32)
        d1 = jnp.zeros((bi, 1), jnp.float32)
        d2 = jnp.zeros((bi, 1), jnp.float32)
        for j, a1, a2 in _unpack_chunks(f_blk, t_blk, self2, i0, bi, w_words):
            d1 = d1 + jnp.sum(a1, axis=1, keepdims=True)
            d2 = d2 + jnp.sum(a2, axis=1, keepdims=True)
        d1_ref[...] = jnp.where(d1 > 0, lax.rsqrt(d1), 0.0)
        d2_ref[...] = jnp.where(d2 > 0, lax.rsqrt(d2), 0.0)

    return pl.pallas_call(
        body,
        grid=(nb,),
        in_specs=[
            pl.BlockSpec((bi, w_words), lambda i: (i, 0)),
            pl.BlockSpec((bi, w_words), lambda i: (i, 0)),
            pl.BlockSpec((bi, 1), lambda i: (i, 0)),
        ],
        out_specs=[
            pl.BlockSpec((bi, 1), lambda i: (i, 0)),
            pl.BlockSpec((bi, 1), lambda i: (i, 0)),
        ],
        out_shape=[
            jax.ShapeDtypeStruct((n_pad, 1), jnp.float32),
            jax.ShapeDtypeStruct((n_pad, 1), jnp.float32),
        ],
        compiler_params=pltpu.CompilerParams(
            dimension_semantics=("parallel",)),
    )(f2d, two2d, selfcnt2d)


def _tc_embed(x_pad, w_embed, d1inv, d2inv, n_pad, bi):
    nb = n_pad // bi
    feat = x_pad.shape[1]
    hid = w_embed.shape[1]

    def body(x_ref, w_ref, d1_ref, d2_ref, r_ref, rv1_ref, rv2_ref):
        r = jnp.maximum(
            jnp.dot(x_ref[...], w_ref[...],
                    preferred_element_type=jnp.float32), 0.0)
        r_ref[...] = r
        rv1_ref[...] = r * d1_ref[...]
        rv2_ref[...] = r * d2_ref[...]

    return pl.pallas_call(
        body,
        grid=(nb,),
        in_specs=[
            pl.BlockSpec((bi, feat), lambda i: (i, 0)),
            pl.BlockSpec((feat, hid), lambda i: (0, 0)),
            pl.BlockSpec((bi, 1), lambda i: (i, 0)),
            pl.BlockSpec((bi, 1), lambda i: (i, 0)),
        ],
        out_specs=[
            pl.BlockSpec((bi, hid), lambda i: (i, 0)),
            pl.BlockSpec((bi, hid), lambda i: (i, 0)),
            pl.BlockSpec((bi, hid), lambda i: (i, 0)),
        ],
        out_shape=[
            jax.ShapeDtypeStruct((n_pad, hid), jnp.float32),
            jax.ShapeDtypeStruct((n_pad, hid), jnp.float32),
            jax.ShapeDtypeStruct((n_pad, hid), jnp.float32),
        ],
        compiler_params=pltpu.CompilerParams(
            dimension_semantics=("parallel",)),
    )(x_pad, w_embed, d1inv, d2inv)


def _tc_hop1(f2d, two2d, selfcnt2d, d1inv, d2inv, rv1, rv2, n_pad, w_words,
             bi):
    nb = n_pad // bi
    hid = rv1.shape[1]

    def body(f_ref, t_ref, sc_ref, d1_ref, d2_ref, v1_ref, v2_ref,
             y_ref, yv1_ref, yv2_ref):
        i0 = pl.program_id(0) * bi
        f_blk = f_ref[...]
        t_blk = t_ref[...]
        self2 = (sc_ref[...] >= 2).astype(jnp.float32)
        acc1 = jnp.zeros((bi, hid), jnp.float32)
        acc2 = jnp.zeros((bi, hid), jnp.float32)
        for j, a1, a2 in _unpack_chunks(f_blk, t_blk, self2, i0, bi, w_words):
            v1 = v1_ref[pl.ds(j * w_words, w_words), :]
            v2 = v2_ref[pl.ds(j * w_words, w_words), :]
            acc1 = acc1 + jnp.dot(a1, v1, preferred_element_type=jnp.float32)
            acc2 = acc2 + jnp.dot(a2, v2, preferred_element_type=jnp.float32)
        d1 = d1_ref[...]
        d2 = d2_ref[...]
        y = jnp.concatenate([acc1 * d1, acc2 * d2], axis=1)
        y_ref[...] = y
        yv1_ref[...] = y * d1
        yv2_ref[...] = y * d2

    return pl.pallas_call(
        body,
        grid=(nb,),
        in_specs=[
            pl.BlockSpec((bi, w_words), lambda i: (i, 0)),
            pl.BlockSpec((bi, w_words), lambda i: (i, 0)),
            pl.BlockSpec((bi, 1), lambda i: (i, 0)),
            pl.BlockSpec((bi, 1), lambda i: (i, 0)),
            pl.BlockSpec((bi, 1), lambda i: (i, 0)),
            pl.BlockSpec((n_pad, hid), lambda i: (0, 0)),
            pl.BlockSpec((n_pad, hid), lambda i: (0, 0)),
        ],
        out_specs=[
            pl.BlockSpec((bi, 2 * hid), lambda i: (i, 0)),
            pl.BlockSpec((bi, 2 * hid), lambda i: (i, 0)),
            pl.BlockSpec((bi, 2 * hid), lambda i: (i, 0)),
        ],
        out_shape=[
            jax.ShapeDtypeStruct((n_pad, 2 * hid), jnp.float32),
            jax.ShapeDtypeStruct((n_pad, 2 * hid), jnp.float32),
            jax.ShapeDtypeStruct((n_pad, 2 * hid), jnp.float32),
        ],
        compiler_params=pltpu.CompilerParams(
            dimension_semantics=("parallel",)),
    )(f2d, two2d, selfcnt2d, d1inv, d2inv, rv1, rv2)


def _tc_hop2_classify(f2d, two2d, selfcnt2d, d1inv, d2inv, yv1, yv2, r, y,
                      w_classify, n_pad, w_words, bi):
    nb = n_pad // bi
    c2 = yv1.shape[1]
    hid = r.shape[1]
    cls = w_classify.shape[1]

    def body(f_ref, t_ref, sc_ref, d1_ref, d2_ref, v1_ref, v2_ref, r_ref,
             y_ref, wc_ref, out_ref):
        i0 = pl.program_id(0) * bi
        f_blk = f_ref[...]
        t_blk = t_ref[...]
        self2 = (sc_ref[...] >= 2).astype(jnp.float32)
        acc1 = jnp.zeros((bi, c2), jnp.float32)
        acc2 = jnp.zeros((bi, c2), jnp.float32)
        for j, a1, a2 in _unpack_chunks(f_blk, t_blk, self2, i0, bi, w_words):
            v1 = v1_ref[pl.ds(j * w_words, w_words), :]
            v2 = v2_ref[pl.ds(j * w_words, w_words), :]
            acc1 = acc1 + jnp.dot(a1, v1, preferred_element_type=jnp.float32)
            acc2 = acc2 + jnp.dot(a2, v2, preferred_element_type=jnp.float32)
        z1 = acc1 * d1_ref[...]
        z2 = acc2 * d2_ref[...]
        out = jnp.dot(r_ref[...], wc_ref[pl.ds(0, hid), :],
                      preferred_element_type=jnp.float32)
        out = out + jnp.dot(y_ref[...], wc_ref[pl.ds(hid, c2), :],
                            preferred_element_type=jnp.float32)
        out = out + jnp.dot(z1, wc_ref[pl.ds(hid + c2, c2), :],
                            preferred_element_type=jnp.float32)
        out = out + jnp.dot(z2, wc_ref[pl.ds(hid + 2 * c2, c2), :],
                            preferred_element_type=jnp.float32)
        out_ref[...] = out

    return pl.pallas_call(
        body,
        grid=(nb,),
        in_specs=[
            pl.BlockSpec((bi, w_words), lambda i: (i, 0)),
            pl.BlockSpec((bi, w_words), lambda i: (i, 0)),
            pl.BlockSpec((bi, 1), lambda i: (i, 0)),
            pl.BlockSpec((bi, 1), lambda i: (i, 0)),
            pl.BlockSpec((bi, 1), lambda i: (i, 0)),
            pl.BlockSpec((n_pad, c2), lambda i: (0, 0)),
            pl.BlockSpec((n_pad, c2), lambda i: (0, 0)),
            pl.BlockSpec((bi, hid), lambda i: (i, 0)),
            pl.BlockSpec((bi, c2), lambda i: (i, 0)),
            pl.BlockSpec(w_classify.shape, lambda i: (0, 0)),
        ],
        out_specs=pl.BlockSpec((bi, cls), lambda i: (i, 0)),
        out_shape=jax.ShapeDtypeStruct((n_pad, cls), jnp.float32),
        compiler_params=pltpu.CompilerParams(
            dimension_semantics=("parallel",)),
    )(f2d, two2d, selfcnt2d, d1inv, d2inv, yv1, yv2, r, y, w_classify)


def _tc_pipeline(f2d, two2d, selfcnt2d, x, w_embed, w_classify, n, n_pad,
                 w_words, bi):
    d1inv, d2inv = _tc_degrees(f2d, two2d, selfcnt2d, n_pad, w_words, bi)

    x_pad = jnp.pad(x, ((0, n_pad - n), (0, 0)))
    r, rv1, rv2 = _tc_embed(x_pad, w_embed, d1inv, d2inv, n_pad, bi)

    y, yv1, yv2 = _tc_hop1(f2d, two2d, selfcnt2d, d1inv, d2inv, rv1, rv2,
                           n_pad, w_words, bi)

    out = _tc_hop2_classify(f2d, two2d, selfcnt2d, d1inv, d2inv, yv1, yv2,
                            r, y, w_classify, n_pad, w_words, bi)
    return out[:n]


def kernel(x, edge_index, w_embed, w_classify):
    n = x.shape[0]
    e = edge_index.shape[1]

    w_words = _round_up(-(-n // 32), 64)
    n_pad = 32 * w_words
    bi = 256

    chunk = 1600
    e_pad = _round_up(e, chunk)

    src = edge_index[0]
    dst = edge_index[1]
    if e_pad != e:
        src = jnp.concatenate(
            [src, jnp.full((e_pad - e,), jnp.int32(2 ** 30))])
        dst = jnp.concatenate([dst, jnp.zeros((e_pad - e,), jnp.int32)])

    f_flat, selfcnt = _sc_build_adj(src, dst, n_pad, w_words, e_pad, chunk)
    f2d = f_flat.reshape(n_pad, w_words)
    g_pitch = _round_up(w_words, 128)
    f2d_g = jnp.pad(f2d, ((0, 0), (0, g_pitch - w_words)))
    two_flat = _sc_two_hop(src, dst, f2d_g, n_pad, w_words, e_pad, chunk)
    two2d = two_flat.reshape(n_pad, w_words)
    selfcnt2d = selfcnt.reshape(n_pad, 1)

    return _tc_pipeline(f2d, two2d, selfcnt2d, x, w_embed, w_classify, n,
                        n_pad, w_words, bi)


# speedup vs baseline: 2.8802x; 1.0317x over previous
import functools

import jax
import jax.numpy as jnp
from jax import lax
from jax.experimental import pallas as pl
from jax.experimental.pallas import tpu as pltpu
from jax.experimental.pallas import tpu_sc as plsc

_NC = 2
_NS = 16
_NWORKERS = _NC * _NS


def _round_up(x, m):
    return (x + m - 1) // m * m


def _sc_build_adj(src, dst, n_pad, w_words, e_pad, chunk):
    n_chunks = e_pad // chunk
    nc_pad = _round_up(n_chunks, 8)
    groups = chunk // 16

    mesh = plsc.VectorSubcoreMesh(core_axis_name="c", subcore_axis_name="s",
                                  num_cores=_NC, num_subcores=_NS)

    @functools.partial(
        pl.kernel,
        mesh=mesh,
        out_type=[
            jax.ShapeDtypeStruct((n_pad * w_words,), jnp.int32),
            jax.ShapeDtypeStruct((n_pad,), jnp.int32),
            jax.ShapeDtypeStruct((_NWORKERS * e_pad,), jnp.int32),
            jax.ShapeDtypeStruct((_NWORKERS * e_pad,), jnp.int32),
            jax.ShapeDtypeStruct((_NWORKERS * nc_pad,), jnp.int32),
        ],
        scratch_types=[
            pltpu.VMEM((chunk,), jnp.int32),
            pltpu.VMEM((chunk,), jnp.int32),
            pltpu.VMEM((chunk + 32,), jnp.int32),
            pltpu.VMEM((chunk + 32,), jnp.int32),
            pltpu.VMEM(((w_words + 1) * w_words,), jnp.int32),
            pltpu.VMEM((w_words + 16,), jnp.int32),
            pltpu.VMEM((nc_pad + 16,), jnp.int32),
            pltpu.SemaphoreType.DMA,
        ],
        compiler_params=pltpu.CompilerParams(needs_layout_passes=False),
    )
    def build(src_hbm, dst_hbm, f_hbm, self_hbm, sl_hbm, dl_hbm, cnt_hbm,
              s_v, d_v, sel_s, sel_d, acc, selfc, cnts, wb_sem):
        wid = lax.axis_index("s") * _NC + lax.axis_index("c")
        base = wid * w_words
        zero16 = jnp.zeros((16,), jnp.int32)
        lane0 = lax.iota(jnp.int32, 16) == 0

        def zero_acc(i, _):
            acc[pl.ds(i * 16, 16)] = zero16
            return 0

        lax.fori_loop(0, (w_words * w_words) // 16, zero_acc, 0)

        def zero_selfc(i, _):
            selfc[pl.ds(i * 16, 16)] = zero16
            return 0

        lax.fori_loop(0, w_words // 16 + 1, zero_selfc, 0)

        def do_chunk(c, _):
            pltpu.sync_copy(src_hbm.at[pl.ds(c * chunk, chunk)], s_v)
            pltpu.sync_copy(dst_hbm.at[pl.ds(c * chunk, chunk)], d_v)

            def compact(g, cnt):
                s16 = s_v[pl.ds(g * 16, 16)]
                d16 = d_v[pl.ds(g * 16, 16)]
                m = jnp.logical_and(s16 >= base, s16 < base + w_words)
                mi = jnp.where(m, 1, 0)
                cs = plsc.cumsum(mi)
                pos = cnt + cs - mi
                plsc.store_scatter(sel_s, [pos], s16 - base, mask=m)
                plsc.store_scatter(sel_d, [pos], d16, mask=m)
                return cnt + cs[15]

            cnt = lax.fori_loop(0, groups, compact, 0)

            cwin = cnts[pl.ds(c, 16)]
            cnts[pl.ds(c, 16)] = jnp.where(lane0, cnt, cwin)
            pltpu.async_copy(sel_s.at[pl.ds(0, chunk)],
                             sl_hbm.at[pl.ds((wid * n_chunks + c) * chunk,
                                             chunk)], wb_sem)
            pltpu.async_copy(sel_d.at[pl.ds(0, chunk)],
                             dl_hbm.at[pl.ds((wid * n_chunks + c) * chunk,
                                             chunk)], wb_sem)

            def scatter_bit(p, _):
                s_loc = sel_s[pl.ds(p, 16)][0]
                dd = sel_d[pl.ds(p, 16)][0]
                wd = lax.rem(dd, w_words)
                bit = lax.div(dd, w_words)
                addr = s_loc * w_words + wd
                win = acc[pl.ds(addr, 16)]
                acc[pl.ds(addr, 16)] = jnp.where(
                    lane0, win | (jnp.int32(1) << bit), win)

                @pl.when(s_loc + base == dd)
                def _():
                    swin = selfc[pl.ds(s_loc, 16)]
                    selfc[pl.ds(s_loc, 16)] = jnp.where(lane0, swin + 1, swin)

                return 0

            lax.fori_loop(0, cnt, scatter_bit, 0)
            pltpu.make_async_copy(
                sel_s.at[pl.ds(0, chunk)],
                sl_hbm.at[pl.ds(wid * chunk, chunk)], wb_sem).wait()
            pltpu.make_async_copy(
                sel_d.at[pl.ds(0, chunk)],
                dl_hbm.at[pl.ds(wid * chunk, chunk)], wb_sem).wait()
            return 0

        lax.fori_loop(0, n_chunks, do_chunk, 0)

        pltpu.sync_copy(acc.at[pl.ds(0, w_words * w_words)],
                        f_hbm.at[pl.ds(base * w_words, w_words * w_words)])
        pltpu.sync_copy(selfc.at[pl.ds(0, w_words)],
                        self_hbm.at[pl.ds(base, w_words)])
        pltpu.sync_copy(cnts.at[pl.ds(0, nc_pad)],
                        cnt_hbm.at[pl.ds(wid * nc_pad, nc_pad)])

    return build(src, dst)


def _sc_two_hop(slist, dlist, counts, f2d, n_pad, w_words, e_pad, chunk):
    n_chunks = e_pad // chunk
    nc_pad = _round_up(n_chunks, 8)
    g_pitch = f2d.shape[1]

    mesh = plsc.VectorSubcoreMesh(core_axis_name="c", subcore_axis_name="s",
                                  num_cores=_NC, num_subcores=_NS)

    @functools.partial(
        pl.kernel,
        mesh=mesh,
        out_type=jax.ShapeDtypeStruct((n_pad * w_words,), jnp.int32),
        scratch_types=[
            pltpu.VMEM((chunk + 48,), jnp.int32),
            pltpu.VMEM((chunk + 48,), jnp.int32),
            pltpu.VMEM((nc_pad + 16,), jnp.int32),
            pltpu.VMEM(((w_words + 1) * w_words,), jnp.int32),
            pltpu.VMEM((16, g_pitch), jnp.int32),
            pltpu.VMEM((16, g_pitch), jnp.int32),
            pltpu.VMEM((16, g_pitch), jnp.int32),
            pltpu.SemaphoreType.DMA,
            pltpu.SemaphoreType.DMA,
            pltpu.SemaphoreType.DMA,
        ],
        compiler_params=pltpu.CompilerParams(needs_layout_passes=False),
    )
    def twohop(sl_hbm, dl_hbm, cnt_hbm, f_hbm, two_hbm, sel_s, sel_d, cnts,
               acc, r0, r1, r2, sm0, sm1, sm2):
        wid = lax.axis_index("s") * _NC + lax.axis_index("c")
        zero16 = jnp.zeros((16,), jnp.int32)
        bufs = ((r0, sm0), (r1, sm1), (r2, sm2))

        def zero_acc(i, _):
            acc[pl.ds(i * 16, 16)] = zero16
            return 0

        lax.fori_loop(0, (w_words * w_words) // 16, zero_acc, 0)

        pltpu.sync_copy(cnt_hbm.at[pl.ds(wid * nc_pad, nc_pad)],
                        cnts.at[pl.ds(0, nc_pad)])

        def issue(g, b):
            rbuf, sem = bufs[b]
            pltpu.async_copy(f_hbm.at[sel_d.at[pl.ds(g * 16, 16)]], rbuf, sem)

        def do_chunk(c, _):
            pltpu.sync_copy(
                sl_hbm.at[pl.ds((wid * n_chunks + c) * chunk, chunk)],
                sel_s.at[pl.ds(0, chunk)])
            pltpu.sync_copy(
                dl_hbm.at[pl.ds((wid * n_chunks + c) * chunk, chunk)],
                sel_d.at[pl.ds(0, chunk)])
            cnt = cnts[pl.ds(c, 16)][0]

            for t in range(3):
                sel_s[pl.ds(cnt + 16 * t, 16)] = jnp.full(
                    (16,), w_words, jnp.int32)
                sel_d[pl.ds(cnt + 16 * t, 16)] = zero16

            ngroups = lax.div(cnt + 15, 16)

            @pl.when(ngroups > 0)
            def _():
                issue(jnp.int32(0), 0)

            @pl.when(ngroups > 1)
            def _():
                issue(jnp.int32(1), 1)

            def or_group(g, b):
                rbuf, sem = bufs[b]
                pltpu.make_async_copy(
                    f_hbm.at[sel_d.at[pl.ds(0, 16)]], rbuf, sem).wait()

                @pl.when(g + 2 < ngroups)
                def _():
                    issue(g + 2, (b + 2) % 3)

                svec = sel_s[pl.ds(g * 16, 16)]
                for l in range(16):
                    s_loc = svec[l]
                    for cc in range(w_words // 16):
                        off = s_loc * w_words + cc * 16
                        acc[pl.ds(off, 16)] = (acc[pl.ds(off, 16)]
                                               | rbuf[l, pl.ds(cc * 16, 16)])

            def do_trip(t, _):
                g = t * 3

                @pl.when(g < ngroups)
                def _():
                    or_group(g, 0)

                @pl.when(g + 1 < ngroups)
                def _():
                    or_group(g + 1, 1)

                @pl.when(g + 2 < ngroups)
                def _():
                    or_group(g + 2, 2)

                return 0

            lax.fori_loop(0, (ngroups + 2) // 3, do_trip, 0)
            return 0

        lax.fori_loop(0, n_chunks, do_chunk, 0)

        pltpu.sync_copy(
            acc.at[pl.ds(0, w_words * w_words)],
            two_hbm.at[pl.ds(wid * w_words * w_words, w_words * w_words)])

    return twohop(slist, dlist, counts, f2d)


def _unpack_chunks(f_blk, two_blk, self2, i0, bi, w_words):
    a2w = two_blk & ~f_blk
    row_ids = i0 + lax.broadcasted_iota(jnp.int32, (bi, w_words), 0)
    col_ids = lax.broadcasted_iota(jnp.int32, (bi, w_words), 1)
    one = jnp.int32(1)
    for j in range(32):
        diag = row_ids == (j * w_words + col_ids)
        a1 = ((f_blk >> j) & one).astype(jnp.float32)
        a2 = ((a2w >> j) & one).astype(jnp.float32)
        a1 = jnp.where(diag, self2, a1)
        a2 = jnp.where(diag, 0.0, a2)
        yield j, a1, a2


def _tc_degrees(f2d, two2d, selfcnt2d, n_pad, w_words, bi):
    nb = n_pad // bi

    def body(f_ref, t_ref, sc_ref, d1_ref, d2_ref):
        i0 = pl.program_id(0) * bi
        f_blk = f_ref[...]
        t_blk = t_ref[...]
        self2 = (sc_ref[...] >= 2).astype(jnp.float32)
        d1 = jnp.zeros((bi, 1), jnp.float32)
        d2 = jnp.zeros((bi, 1), jnp.float32)
        for j, a1, a2 in _unpack_chunks(f_blk, t_blk, self2, i0, bi, w_words):
            d1 = d1 + jnp.sum(a1, axis=1, keepdims=True)
            d2 = d2 + jnp.sum(a2, axis=1, keepdims=True)
        d1_ref[...] = jnp.where(d1 > 0, lax.rsqrt(d1), 0.0)
        d2_ref[...] = jnp.where(d2 > 0, lax.rsqrt(d2), 0.0)

    return pl.pallas_call(
        body,
        grid=(nb,),
        in_specs=[
            pl.BlockSpec((bi, w_words), lambda i: (i, 0)),
            pl.BlockSpec((bi, w_words), lambda i: (i, 0)),
            pl.BlockSpec((bi, 1), lambda i: (i, 0)),
        ],
        out_specs=[
            pl.BlockSpec((bi, 1), lambda i: (i, 0)),
            pl.BlockSpec((bi, 1), lambda i: (i, 0)),
        ],
        out_shape=[
            jax.ShapeDtypeStruct((n_pad, 1), jnp.float32),
            jax.ShapeDtypeStruct((n_pad, 1), jnp.float32),
        ],
        compiler_params=pltpu.CompilerParams(
            dimension_semantics=("parallel",)),
    )(f2d, two2d, selfcnt2d)


def _tc_embed(x_pad, w_embed, d1inv, d2inv, n_pad, bi):
    nb = n_pad // bi
    feat = x_pad.shape[1]
    hid = w_embed.shape[1]

    def body(x_ref, w_ref, d1_ref, d2_ref, r_ref, rv1_ref, rv2_ref):
        r = jnp.maximum(
            jnp.dot(x_ref[...], w_ref[...],
                    preferred_element_type=jnp.float32), 0.0)
        r_ref[...] = r
        rv1_ref[...] = r * d1_ref[...]
        rv2_ref[...] = r * d2_ref[...]

    return pl.pallas_call(
        body,
        grid=(nb,),
        in_specs=[
            pl.BlockSpec((bi, feat), lambda i: (i, 0)),
            pl.BlockSpec((feat, hid), lambda i: (0, 0)),
            pl.BlockSpec((bi, 1), lambda i: (i, 0)),
            pl.BlockSpec((bi, 1), lambda i: (i, 0)),
        ],
        out_specs=[
            pl.BlockSpec((bi, hid), lambda i: (i, 0)),
            pl.BlockSpec((bi, hid), lambda i: (i, 0)),
            pl.BlockSpec((bi, hid), lambda i: (i, 0)),
        ],
        out_shape=[
            jax.ShapeDtypeStruct((n_pad, hid), jnp.float32),
            jax.ShapeDtypeStruct((n_pad, hid), jnp.float32),
            jax.ShapeDtypeStruct((n_pad, hid), jnp.float32),
        ],
        compiler_params=pltpu.CompilerParams(
            dimension_semantics=("parallel",)),
    )(x_pad, w_embed, d1inv, d2inv)


def _tc_hop1(f2d, two2d, selfcnt2d, d1inv, d2inv, rv1, rv2, n_pad, w_words,
             bi):
    nb = n_pad // bi
    hid = rv1.shape[1]

    def body(f_ref, t_ref, sc_ref, d1_ref, d2_ref, v1_ref, v2_ref,
             y_ref, yv1_ref, yv2_ref):
        i0 = pl.program_id(0) * bi
        f_blk = f_ref[...]
        t_blk = t_ref[...]
        self2 = (sc_ref[...] >= 2).astype(jnp.float32)
        acc1 = jnp.zeros((bi, hid), jnp.float32)
        acc2 = jnp.zeros((bi, hid), jnp.float32)
        for j, a1, a2 in _unpack_chunks(f_blk, t_blk, self2, i0, bi, w_words):
            v1 = v1_ref[pl.ds(j * w_words, w_words), :]
            v2 = v2_ref[pl.ds(j * w_words, w_words), :]
            acc1 = acc1 + jnp.dot(a1, v1, preferred_element_type=jnp.float32)
            acc2 = acc2 + jnp.dot(a2, v2, preferred_element_type=jnp.float32)
        d1 = d1_ref[...]
        d2 = d2_ref[...]
        y = jnp.concatenate([acc1 * d1, acc2 * d2], axis=1)
        y_ref[...] = y
        yv1_ref[...] = y * d1
        yv2_ref[...] = y * d2

    return pl.pallas_call(
        body,
        grid=(nb,),
        in_specs=[
            pl.BlockSpec((bi, w_words), lambda i: (i, 0)),
            pl.BlockSpec((bi, w_words), lambda i: (i, 0)),
            pl.BlockSpec((bi, 1), lambda i: (i, 0)),
            pl.BlockSpec((bi, 1), lambda i: (i, 0)),
            pl.BlockSpec((bi, 1), lambda i: (i, 0)),
            pl.BlockSpec((n_pad, hid), lambda i: (0, 0)),
            pl.BlockSpec((n_pad, hid), lambda i: (0, 0)),
        ],
        out_specs=[
            pl.BlockSpec((bi, 2 * hid), lambda i: (i, 0)),
            pl.BlockSpec((bi, 2 * hid), lambda i: (i, 0)),
            pl.BlockSpec((bi, 2 * hid), lambda i: (i, 0)),
        ],
        out_shape=[
            jax.ShapeDtypeStruct((n_pad, 2 * hid), jnp.float32),
            jax.ShapeDtypeStruct((n_pad, 2 * hid), jnp.float32),
            jax.ShapeDtypeStruct((n_pad, 2 * hid), jnp.float32),
        ],
        compiler_params=pltpu.CompilerParams(
            dimension_semantics=("parallel",)),
    )(f2d, two2d, selfcnt2d, d1inv, d2inv, rv1, rv2)


def _tc_hop2_classify(f2d, two2d, selfcnt2d, d1inv, d2inv, yv1, yv2, r, y,
                      w_classify, n_pad, w_words, bi):
    nb = n_pad // bi
    c2 = yv1.shape[1]
    hid = r.shape[1]
    cls = w_classify.shape[1]

    def body(f_ref, t_ref, sc_ref, d1_ref, d2_ref, v1_ref, v2_ref, r_ref,
             y_ref, wc_ref, out_ref):
        i0 = pl.program_id(0) * bi
        f_blk = f_ref[...]
        t_blk = t_ref[...]
        self2 = (sc_ref[...] >= 2).astype(jnp.float32)
        acc1 = jnp.zeros((bi, c2), jnp.float32)
        acc2 = jnp.zeros((bi, c2), jnp.float32)
        for j, a1, a2 in _unpack_chunks(f_blk, t_blk, self2, i0, bi, w_words):
            v1 = v1_ref[pl.ds(j * w_words, w_words), :]
            v2 = v2_ref[pl.ds(j * w_words, w_words), :]
            acc1 = acc1 + jnp.dot(a1, v1, preferred_element_type=jnp.float32)
            acc2 = acc2 + jnp.dot(a2, v2, preferred_element_type=jnp.float32)
        z1 = acc1 * d1_ref[...]
        z2 = acc2 * d2_ref[...]
        out = jnp.dot(r_ref[...], wc_ref[pl.ds(0, hid), :],
                      preferred_element_type=jnp.float32)
        out = out + jnp.dot(y_ref[...], wc_ref[pl.ds(hid, c2), :],
                            preferred_element_type=jnp.float32)
        out = out + jnp.dot(z1, wc_ref[pl.ds(hid + c2, c2), :],
                            preferred_element_type=jnp.float32)
        out = out + jnp.dot(z2, wc_ref[pl.ds(hid + 2 * c2, c2), :],
                            preferred_element_type=jnp.float32)
        out_ref[...] = out

    return pl.pallas_call(
        body,
        grid=(nb,),
        in_specs=[
            pl.BlockSpec((bi, w_words), lambda i: (i, 0)),
            pl.BlockSpec((bi, w_words), lambda i: (i, 0)),
            pl.BlockSpec((bi, 1), lambda i: (i, 0)),
            pl.BlockSpec((bi, 1), lambda i: (i, 0)),
            pl.BlockSpec((bi, 1), lambda i: (i, 0)),
            pl.BlockSpec((n_pad, c2), lambda i: (0, 0)),
            pl.BlockSpec((n_pad, c2), lambda i: (0, 0)),
            pl.BlockSpec((bi, hid), lambda i: (i, 0)),
            pl.BlockSpec((bi, c2), lambda i: (i, 0)),
            pl.BlockSpec(w_classify.shape, lambda i: (0, 0)),
        ],
        out_specs=pl.BlockSpec((bi, cls), lambda i: (i, 0)),
        out_shape=jax.ShapeDtypeStruct((n_pad, cls), jnp.float32),
        compiler_params=pltpu.CompilerParams(
            dimension_semantics=("parallel",)),
    )(f2d, two2d, selfcnt2d, d1inv, d2inv, yv1, yv2, r, y, w_classify)


def _tc_pipeline(f2d, two2d, selfcnt2d, x, w_embed, w_classify, n, n_pad,
                 w_words, bi):
    d1inv, d2inv = _tc_degrees(f2d, two2d, selfcnt2d, n_pad, w_words, bi)

    x_pad = jnp.pad(x, ((0, n_pad - n), (0, 0)))
    r, rv1, rv2 = _tc_embed(x_pad, w_embed, d1inv, d2inv, n_pad, bi)

    y, yv1, yv2 = _tc_hop1(f2d, two2d, selfcnt2d, d1inv, d2inv, rv1, rv2,
                           n_pad, w_words, bi)

    out = _tc_hop2_classify(f2d, two2d, selfcnt2d, d1inv, d2inv, yv1, yv2,
                            r, y, w_classify, n_pad, w_words, bi)
    return out[:n]


def kernel(x, edge_index, w_embed, w_classify):
    n = x.shape[0]
    e = edge_index.shape[1]

    w_words = _round_up(-(-n // 32), 64)
    n_pad = 32 * w_words
    bi = 256

    chunk = 1600
    e_pad = _round_up(e, chunk)

    src = edge_index[0]
    dst = edge_index[1]
    if e_pad != e:
        src = jnp.concatenate(
            [src, jnp.full((e_pad - e,), jnp.int32(2 ** 30))])
        dst = jnp.concatenate([dst, jnp.zeros((e_pad - e,), jnp.int32)])

    f_flat, selfcnt, slist, dlist, counts = _sc_build_adj(
        src, dst, n_pad, w_words, e_pad, chunk)
    f2d = f_flat.reshape(n_pad, w_words)
    g_pitch = _round_up(w_words, 128)
    f2d_g = jnp.pad(f2d, ((0, 0), (0, g_pitch - w_words)))
    two_flat = _sc_two_hop(slist, dlist, counts, f2d_g, n_pad, w_words,
                           e_pad, chunk)
    two2d = two_flat.reshape(n_pad, w_words)
    selfcnt2d = selfcnt.reshape(n_pad, 1)

    return _tc_pipeline(f2d, two2d, selfcnt2d, x, w_embed, w_classify, n,
                        n_pad, w_words, bi)


# speedup vs baseline: 3.0164x; 1.0473x over previous
import functools

import jax
import jax.numpy as jnp
from jax import lax
from jax.experimental import pallas as pl
from jax.experimental.pallas import tpu as pltpu
from jax.experimental.pallas import tpu_sc as plsc

_NC = 2
_NS = 16
_NWORKERS = _NC * _NS


def _round_up(x, m):
    return (x + m - 1) // m * m


def _sc_build_adj(src, dst, n_pad, w_words, e_pad, chunk):
    n_chunks = e_pad // chunk
    nc_pad = _round_up(n_chunks, 8)
    groups = chunk // 16

    mesh = plsc.VectorSubcoreMesh(core_axis_name="c", subcore_axis_name="s",
                                  num_cores=_NC, num_subcores=_NS)

    @functools.partial(
        pl.kernel,
        mesh=mesh,
        out_type=[
            jax.ShapeDtypeStruct((n_pad * w_words,), jnp.int32),
            jax.ShapeDtypeStruct((n_pad,), jnp.int32),
            jax.ShapeDtypeStruct((_NWORKERS * e_pad,), jnp.int32),
            jax.ShapeDtypeStruct((_NWORKERS * e_pad,), jnp.int32),
            jax.ShapeDtypeStruct((_NWORKERS * nc_pad,), jnp.int32),
        ],
        scratch_types=[
            pltpu.VMEM((chunk,), jnp.int32),
            pltpu.VMEM((chunk,), jnp.int32),
            pltpu.VMEM((chunk + 32,), jnp.int32),
            pltpu.VMEM((chunk + 32,), jnp.int32),
            pltpu.VMEM(((w_words + 1) * w_words,), jnp.int32),
            pltpu.VMEM((w_words + 16,), jnp.int32),
            pltpu.VMEM((nc_pad + 16,), jnp.int32),
            pltpu.SemaphoreType.DMA,
        ],
        compiler_params=pltpu.CompilerParams(needs_layout_passes=False),
    )
    def build(src_hbm, dst_hbm, f_hbm, self_hbm, sl_hbm, dl_hbm, cnt_hbm,
              s_v, d_v, sel_s, sel_d, acc, selfc, cnts, wb_sem):
        wid = lax.axis_index("s") * _NC + lax.axis_index("c")
        base = wid * w_words
        zero16 = jnp.zeros((16,), jnp.int32)
        lane0 = lax.iota(jnp.int32, 16) == 0

        def zero_acc(i, _):
            acc[pl.ds(i * 16, 16)] = zero16
            return 0

        lax.fori_loop(0, (w_words * w_words) // 16, zero_acc, 0)

        def zero_selfc(i, _):
            selfc[pl.ds(i * 16, 16)] = zero16
            return 0

        lax.fori_loop(0, w_words // 16 + 1, zero_selfc, 0)

        def do_chunk(c, _):
            pltpu.sync_copy(src_hbm.at[pl.ds(c * chunk, chunk)], s_v)
            pltpu.sync_copy(dst_hbm.at[pl.ds(c * chunk, chunk)], d_v)

            def compact(g, cnt):
                s16 = s_v[pl.ds(g * 16, 16)]
                d16 = d_v[pl.ds(g * 16, 16)]
                m = jnp.logical_and(s16 >= base, s16 < base + w_words)
                mi = jnp.where(m, 1, 0)
                cs = plsc.cumsum(mi)
                pos = cnt + cs - mi
                plsc.store_scatter(sel_s, [pos], s16 - base, mask=m)
                plsc.store_scatter(sel_d, [pos], d16, mask=m)
                return cnt + cs[15]

            cnt = lax.fori_loop(0, groups, compact, 0)

            cwin = cnts[pl.ds(c, 16)]
            cnts[pl.ds(c, 16)] = jnp.where(lane0, cnt, cwin)
            pltpu.async_copy(sel_s.at[pl.ds(0, chunk)],
                             sl_hbm.at[pl.ds((wid * n_chunks + c) * chunk,
                                             chunk)], wb_sem)
            pltpu.async_copy(sel_d.at[pl.ds(0, chunk)],
                             dl_hbm.at[pl.ds((wid * n_chunks + c) * chunk,
                                             chunk)], wb_sem)

            def scatter_bit(p, _):
                s_loc = sel_s[pl.ds(p, 16)][0]
                dd = sel_d[pl.ds(p, 16)][0]
                wd = lax.rem(dd, w_words)
                bit = lax.div(dd, w_words)
                addr = s_loc * w_words + wd
                win = acc[pl.ds(addr, 16)]
                acc[pl.ds(addr, 16)] = jnp.where(
                    lane0, win | (jnp.int32(1) << bit), win)

                @pl.when(s_loc + base == dd)
                def _():
                    swin = selfc[pl.ds(s_loc, 16)]
                    selfc[pl.ds(s_loc, 16)] = jnp.where(lane0, swin + 1, swin)

                return 0

            lax.fori_loop(0, cnt, scatter_bit, 0)
            pltpu.make_async_copy(
                sel_s.at[pl.ds(0, chunk)],
                sl_hbm.at[pl.ds(wid * chunk, chunk)], wb_sem).wait()
            pltpu.make_async_copy(
                sel_d.at[pl.ds(0, chunk)],
                dl_hbm.at[pl.ds(wid * chunk, chunk)], wb_sem).wait()
            return 0

        lax.fori_loop(0, n_chunks, do_chunk, 0)

        pltpu.sync_copy(acc.at[pl.ds(0, w_words * w_words)],
                        f_hbm.at[pl.ds(base * w_words, w_words * w_words)])
        pltpu.sync_copy(selfc.at[pl.ds(0, w_words)],
                        self_hbm.at[pl.ds(base, w_words)])
        pltpu.sync_copy(cnts.at[pl.ds(0, nc_pad)],
                        cnt_hbm.at[pl.ds(wid * nc_pad, nc_pad)])

    return build(src, dst)


def _sc_two_hop(slist, dlist, counts, f2d, n_pad, w_words, e_pad, chunk):
    n_chunks = e_pad // chunk
    nc_pad = _round_up(n_chunks, 8)
    g_pitch = f2d.shape[1]

    mesh = plsc.VectorSubcoreMesh(core_axis_name="c", subcore_axis_name="s",
                                  num_cores=_NC, num_subcores=_NS)

    @functools.partial(
        pl.kernel,
        mesh=mesh,
        out_type=jax.ShapeDtypeStruct((n_pad * w_words,), jnp.int32),
        scratch_types=[
            pltpu.VMEM((chunk + 48,), jnp.int32),
            pltpu.VMEM((chunk + 48,), jnp.int32),
            pltpu.VMEM((nc_pad + 16,), jnp.int32),
            pltpu.VMEM(((w_words + 1) * w_words,), jnp.int32),
            pltpu.VMEM((16, g_pitch), jnp.int32),
            pltpu.VMEM((16, g_pitch), jnp.int32),
            pltpu.VMEM((16, g_pitch), jnp.int32),
            pltpu.SemaphoreType.DMA,
            pltpu.SemaphoreType.DMA,
            pltpu.SemaphoreType.DMA,
        ],
        compiler_params=pltpu.CompilerParams(needs_layout_passes=False),
    )
    def twohop(sl_hbm, dl_hbm, cnt_hbm, f_hbm, two_hbm, sel_s, sel_d, cnts,
               acc, r0, r1, r2, sm0, sm1, sm2):
        wid = lax.axis_index("s") * _NC + lax.axis_index("c")
        zero16 = jnp.zeros((16,), jnp.int32)
        bufs = ((r0, sm0), (r1, sm1), (r2, sm2))

        def zero_acc(i, _):
            acc[pl.ds(i * 16, 16)] = zero16
            return 0

        lax.fori_loop(0, (w_words * w_words) // 16, zero_acc, 0)

        pltpu.sync_copy(cnt_hbm.at[pl.ds(wid * nc_pad, nc_pad)],
                        cnts.at[pl.ds(0, nc_pad)])

        def issue(g, b):
            rbuf, sem = bufs[b]
            pltpu.async_copy(f_hbm.at[sel_d.at[pl.ds(g * 16, 16)]], rbuf, sem)

        def do_chunk(c, _):
            pltpu.sync_copy(
                sl_hbm.at[pl.ds((wid * n_chunks + c) * chunk, chunk)],
                sel_s.at[pl.ds(0, chunk)])
            pltpu.sync_copy(
                dl_hbm.at[pl.ds((wid * n_chunks + c) * chunk, chunk)],
                sel_d.at[pl.ds(0, chunk)])
            cnt = cnts[pl.ds(c, 16)][0]

            for t in range(3):
                sel_s[pl.ds(cnt + 16 * t, 16)] = jnp.full(
                    (16,), w_words, jnp.int32)
                sel_d[pl.ds(cnt + 16 * t, 16)] = zero16

            ngroups = lax.div(cnt + 15, 16)

            @pl.when(ngroups > 0)
            def _():
                issue(jnp.int32(0), 0)

            @pl.when(ngroups > 1)
            def _():
                issue(jnp.int32(1), 1)

            def or_group(g, b):
                rbuf, sem = bufs[b]
                pltpu.make_async_copy(
                    f_hbm.at[sel_d.at[pl.ds(0, 16)]], rbuf, sem).wait()

                @pl.when(g + 2 < ngroups)
                def _():
                    issue(g + 2, (b + 2) % 3)

                svec = sel_s[pl.ds(g * 16, 16)]
                for l in range(16):
                    s_loc = svec[l]
                    for cc in range(w_words // 16):
                        off = s_loc * w_words + cc * 16
                        acc[pl.ds(off, 16)] = (acc[pl.ds(off, 16)]
                                               | rbuf[l, pl.ds(cc * 16, 16)])

            def do_trip(t, _):
                g = t * 3

                @pl.when(g < ngroups)
                def _():
                    or_group(g, 0)

                @pl.when(g + 1 < ngroups)
                def _():
                    or_group(g + 1, 1)

                @pl.when(g + 2 < ngroups)
                def _():
                    or_group(g + 2, 2)

                return 0

            lax.fori_loop(0, (ngroups + 2) // 3, do_trip, 0)
            return 0

        lax.fori_loop(0, n_chunks, do_chunk, 0)

        pltpu.sync_copy(
            acc.at[pl.ds(0, w_words * w_words)],
            two_hbm.at[pl.ds(wid * w_words * w_words, w_words * w_words)])

    return twohop(slist, dlist, counts, f2d)


def _unpack_chunks(f_blk, a2w, bi, w_words):
    one = jnp.int32(1)
    for j in range(32):
        a1 = ((f_blk >> j) & one).astype(jnp.float32)
        a2 = ((a2w >> j) & one).astype(jnp.float32)
        yield j, a1, a2


def _diag_terms(f_blk, a2w, sc_blk, i0, bi, w_words):
    rows = i0 + lax.broadcasted_iota(jnp.int32, (bi, w_words), 0)
    cols = lax.broadcasted_iota(jnp.int32, (bi, w_words), 1)
    sel = cols == lax.rem(rows, w_words)
    fw = jnp.sum(jnp.where(sel, f_blk, 0), axis=1, keepdims=True)
    aw = jnp.sum(jnp.where(sel, a2w, 0), axis=1, keepdims=True)
    bitpos = lax.div(i0 + lax.broadcasted_iota(jnp.int32, (bi, 1), 0),
                     w_words)
    one = jnp.int32(1)
    d1 = ((fw >> bitpos) & one).astype(jnp.float32)
    d2 = ((aw >> bitpos) & one).astype(jnp.float32)
    self2 = (sc_blk >= 2).astype(jnp.float32)
    return self2 - d1, -d2


def _tc_degrees(f2d, two2d, selfcnt2d, r, n_pad, w_words, bi):
    nb = n_pad // bi
    hid = r.shape[1]

    def body(f_ref, t_ref, sc_ref, r_ref, d1_ref, d2_ref, rv1_ref, rv2_ref):
        i0 = pl.program_id(0) * bi
        f_blk = f_ref[...]
        a2w = t_ref[...] & ~f_blk
        d1 = jnp.zeros((bi, 1), jnp.float32)
        d2 = jnp.zeros((bi, 1), jnp.float32)
        for j, a1, a2 in _unpack_chunks(f_blk, a2w, bi, w_words):
            d1 = d1 + jnp.sum(a1, axis=1, keepdims=True)
            d2 = d2 + jnp.sum(a2, axis=1, keepdims=True)
        c1, c2 = _diag_terms(f_blk, a2w, sc_ref[...], i0, bi, w_words)
        d1 = d1 + c1
        d2 = d2 + c2
        d1 = jnp.where(d1 > 0, lax.rsqrt(d1), 0.0)
        d2 = jnp.where(d2 > 0, lax.rsqrt(d2), 0.0)
        d1_ref[...] = d1
        d2_ref[...] = d2
        rv1_ref[...] = r_ref[...] * d1
        rv2_ref[...] = r_ref[...] * d2

    return pl.pallas_call(
        body,
        grid=(nb,),
        in_specs=[
            pl.BlockSpec((bi, w_words), lambda i: (i, 0)),
            pl.BlockSpec((bi, w_words), lambda i: (i, 0)),
            pl.BlockSpec((bi, 1), lambda i: (i, 0)),
            pl.BlockSpec((bi, hid), lambda i: (i, 0)),
        ],
        out_specs=[
            pl.BlockSpec((bi, 1), lambda i: (i, 0)),
            pl.BlockSpec((bi, 1), lambda i: (i, 0)),
            pl.BlockSpec((bi, hid), lambda i: (i, 0)),
            pl.BlockSpec((bi, hid), lambda i: (i, 0)),
        ],
        out_shape=[
            jax.ShapeDtypeStruct((n_pad, 1), jnp.float32),
            jax.ShapeDtypeStruct((n_pad, 1), jnp.float32),
            jax.ShapeDtypeStruct((n_pad, hid), jnp.float32),
            jax.ShapeDtypeStruct((n_pad, hid), jnp.float32),
        ],
        compiler_params=pltpu.CompilerParams(
            dimension_semantics=("parallel",)),
    )(f2d, two2d, selfcnt2d, r)


def _tc_embed(x_pad, w_embed, n_pad, bi):
    nb = n_pad // bi
    feat = x_pad.shape[1]
    hid = w_embed.shape[1]

    def body(x_ref, w_ref, r_ref):
        r_ref[...] = jnp.maximum(
            jnp.dot(x_ref[...], w_ref[...],
                    preferred_element_type=jnp.float32), 0.0)

    return pl.pallas_call(
        body,
        grid=(nb,),
        in_specs=[
            pl.BlockSpec((bi, feat), lambda i: (i, 0)),
            pl.BlockSpec((feat, hid), lambda i: (0, 0)),
        ],
        out_specs=pl.BlockSpec((bi, hid), lambda i: (i, 0)),
        out_shape=jax.ShapeDtypeStruct((n_pad, hid), jnp.float32),
        compiler_params=pltpu.CompilerParams(
            dimension_semantics=("parallel",)),
    )(x_pad, w_embed)


def _tc_hop1(f2d, two2d, selfcnt2d, d1inv, d2inv, rv1, rv2, n_pad, w_words,
             bi):
    nb = n_pad // bi
    hid = rv1.shape[1]

    def body(f_ref, t_ref, sc_ref, d1_ref, d2_ref, v1_ref, v2_ref,
             y_ref, yv1_ref, yv2_ref):
        i0 = pl.program_id(0) * bi
        f_blk = f_ref[...]
        a2w = t_ref[...] & ~f_blk
        acc1 = jnp.zeros((bi, hid), jnp.float32)
        acc2 = jnp.zeros((bi, hid), jnp.float32)
        for j, a1, a2 in _unpack_chunks(f_blk, a2w, bi, w_words):
            v1 = v1_ref[pl.ds(j * w_words, w_words), :]
            v2 = v2_ref[pl.ds(j * w_words, w_words), :]
            acc1 = acc1 + jnp.dot(a1, v1, preferred_element_type=jnp.float32)
            acc2 = acc2 + jnp.dot(a2, v2, preferred_element_type=jnp.float32)
        c1, c2 = _diag_terms(f_blk, a2w, sc_ref[...], i0, bi, w_words)
        acc1 = acc1 + c1 * v1_ref[pl.ds(i0, bi), :]
        acc2 = acc2 + c2 * v2_ref[pl.ds(i0, bi), :]
        d1 = d1_ref[...]
        d2 = d2_ref[...]
        y = jnp.concatenate([acc1 * d1, acc2 * d2], axis=1)
        y_ref[...] = y
        yv1_ref[...] = y * d1
        yv2_ref[...] = y * d2

    return pl.pallas_call(
        body,
        grid=(nb,),
        in_specs=[
            pl.BlockSpec((bi, w_words), lambda i: (i, 0)),
            pl.BlockSpec((bi, w_words), lambda i: (i, 0)),
            pl.BlockSpec((bi, 1), lambda i: (i, 0)),
            pl.BlockSpec((bi, 1), lambda i: (i, 0)),
            pl.BlockSpec((bi, 1), lambda i: (i, 0)),
            pl.BlockSpec((n_pad, hid), lambda i: (0, 0)),
            pl.BlockSpec((n_pad, hid), lambda i: (0, 0)),
        ],
        out_specs=[
            pl.BlockSpec((bi, 2 * hid), lambda i: (i, 0)),
            pl.BlockSpec((bi, 2 * hid), lambda i: (i, 0)),
            pl.BlockSpec((bi, 2 * hid), lambda i: (i, 0)),
        ],
        out_shape=[
            jax.ShapeDtypeStruct((n_pad, 2 * hid), jnp.float32),
            jax.ShapeDtypeStruct((n_pad, 2 * hid), jnp.float32),
            jax.ShapeDtypeStruct((n_pad, 2 * hid), jnp.float32),
        ],
        compiler_params=pltpu.CompilerParams(
            dimension_semantics=("parallel",)),
    )(f2d, two2d, selfcnt2d, d1inv, d2inv, rv1, rv2)


def _tc_hop2_classify(f2d, two2d, selfcnt2d, d1inv, d2inv, yv1, yv2, r, y,
                      w_classify, n_pad, w_words, bi):
    nb = n_pad // bi
    c2 = yv1.shape[1]
    hid = r.shape[1]
    cls = w_classify.shape[1]

    def body(f_ref, t_ref, sc_ref, d1_ref, d2_ref, v1_ref, v2_ref, r_ref,
             y_ref, wc_ref, out_ref):
        i0 = pl.program_id(0) * bi
        f_blk = f_ref[...]
        a2w = t_ref[...] & ~f_blk
        acc1 = jnp.zeros((bi, c2), jnp.float32)
        acc2 = jnp.zeros((bi, c2), jnp.float32)
        for j, a1, a2 in _unpack_chunks(f_blk, a2w, bi, w_words):
            v1 = v1_ref[pl.ds(j * w_words, w_words), :]
            v2 = v2_ref[pl.ds(j * w_words, w_words), :]
            acc1 = acc1 + jnp.dot(a1, v1, preferred_element_type=jnp.float32)
            acc2 = acc2 + jnp.dot(a2, v2, preferred_element_type=jnp.float32)
        cc1, cc2 = _diag_terms(f_blk, a2w, sc_ref[...], i0, bi, w_words)
        acc1 = acc1 + cc1 * v1_ref[pl.ds(i0, bi), :]
        acc2 = acc2 + cc2 * v2_ref[pl.ds(i0, bi), :]
        z1 = acc1 * d1_ref[...]
        z2 = acc2 * d2_ref[...]
        out = jnp.dot(r_ref[...], wc_ref[pl.ds(0, hid), :],
                      preferred_element_type=jnp.float32)
        out = out + jnp.dot(y_ref[...], wc_ref[pl.ds(hid, c2), :],
                            preferred_element_type=jnp.float32)
        out = out + jnp.dot(z1, wc_ref[pl.ds(hid + c2, c2), :],
                            preferred_element_type=jnp.float32)
        out = out + jnp.dot(z2, wc_ref[pl.ds(hid + 2 * c2, c2), :],
                            preferred_element_type=jnp.float32)
        out_ref[...] = out

    return pl.pallas_call(
        body,
        grid=(nb,),
        in_specs=[
            pl.BlockSpec((bi, w_words), lambda i: (i, 0)),
            pl.BlockSpec((bi, w_words), lambda i: (i, 0)),
            pl.BlockSpec((bi, 1), lambda i: (i, 0)),
            pl.BlockSpec((bi, 1), lambda i: (i, 0)),
            pl.BlockSpec((bi, 1), lambda i: (i, 0)),
            pl.BlockSpec((n_pad, c2), lambda i: (0, 0)),
            pl.BlockSpec((n_pad, c2), lambda i: (0, 0)),
            pl.BlockSpec((bi, hid), lambda i: (i, 0)),
            pl.BlockSpec((bi, c2), lambda i: (i, 0)),
            pl.BlockSpec(w_classify.shape, lambda i: (0, 0)),
        ],
        out_specs=pl.BlockSpec((bi, cls), lambda i: (i, 0)),
        out_shape=jax.ShapeDtypeStruct((n_pad, cls), jnp.float32),
        compiler_params=pltpu.CompilerParams(
            dimension_semantics=("parallel",)),
    )(f2d, two2d, selfcnt2d, d1inv, d2inv, yv1, yv2, r, y, w_classify)


def _tc_pipeline(f2d, two2d, selfcnt2d, x, w_embed, w_classify, n, n_pad,
                 w_words, bi):
    x_pad = jnp.pad(x, ((0, n_pad - n), (0, 0)))
    r = _tc_embed(x_pad, w_embed, n_pad, bi)

    d1inv, d2inv, rv1, rv2 = _tc_degrees(f2d, two2d, selfcnt2d, r, n_pad,
                                         w_words, bi)

    y, yv1, yv2 = _tc_hop1(f2d, two2d, selfcnt2d, d1inv, d2inv, rv1, rv2,
                           n_pad, w_words, bi)

    out = _tc_hop2_classify(f2d, two2d, selfcnt2d, d1inv, d2inv, yv1, yv2,
                            r, y, w_classify, n_pad, w_words, bi)
    return out[:n]


def kernel(x, edge_index, w_embed, w_classify):
    n = x.shape[0]
    e = edge_index.shape[1]

    w_words = _round_up(-(-n // 32), 64)
    n_pad = 32 * w_words
    bi = 256

    chunk = 1600
    e_pad = _round_up(e, chunk)

    src = edge_index[0]
    dst = edge_index[1]
    if e_pad != e:
        src = jnp.concatenate(
            [src, jnp.full((e_pad - e,), jnp.int32(2 ** 30))])
        dst = jnp.concatenate([dst, jnp.zeros((e_pad - e,), jnp.int32)])

    f_flat, selfcnt, slist, dlist, counts = _sc_build_adj(
        src, dst, n_pad, w_words, e_pad, chunk)
    f2d = f_flat.reshape(n_pad, w_words)
    g_pitch = _round_up(w_words, 128)
    f2d_g = jnp.pad(f2d, ((0, 0), (0, g_pitch - w_words)))
    two_flat = _sc_two_hop(slist, dlist, counts, f2d_g, n_pad, w_words,
                           e_pad, chunk)
    two2d = two_flat.reshape(n_pad, w_words)
    selfcnt2d = selfcnt.reshape(n_pad, 1)

    return _tc_pipeline(f2d, two2d, selfcnt2d, x, w_embed, w_classify, n,
                        n_pad, w_words, bi)


# speedup vs baseline: 3.0191x; 1.0009x over previous
import functools

import jax
import jax.numpy as jnp
from jax import lax
from jax.experimental import pallas as pl
from jax.experimental.pallas import tpu as pltpu
from jax.experimental.pallas import tpu_sc as plsc

_NC = 2
_NS = 16
_NWORKERS = _NC * _NS


def _round_up(x, m):
    return (x + m - 1) // m * m


def _sc_build_adj(src, dst, n_pad, w_words, e_pad, chunk):
    n_chunks = e_pad // chunk
    nc_pad = _round_up(n_chunks, 8)
    groups = chunk // 16

    mesh = plsc.VectorSubcoreMesh(core_axis_name="c", subcore_axis_name="s",
                                  num_cores=_NC, num_subcores=_NS)

    @functools.partial(
        pl.kernel,
        mesh=mesh,
        out_type=[
            jax.ShapeDtypeStruct((n_pad * w_words,), jnp.int32),
            jax.ShapeDtypeStruct((n_pad,), jnp.int32),
            jax.ShapeDtypeStruct((_NWORKERS * e_pad,), jnp.int32),
            jax.ShapeDtypeStruct((_NWORKERS * e_pad,), jnp.int32),
            jax.ShapeDtypeStruct((_NWORKERS * nc_pad,), jnp.int32),
        ],
        scratch_types=[
            pltpu.VMEM((chunk,), jnp.int32),
            pltpu.VMEM((chunk,), jnp.int32),
            pltpu.VMEM((chunk + 32,), jnp.int32),
            pltpu.VMEM((chunk + 32,), jnp.int32),
            pltpu.VMEM(((w_words + 1) * w_words,), jnp.int32),
            pltpu.VMEM((w_words + 16,), jnp.int32),
            pltpu.VMEM((nc_pad + 16,), jnp.int32),
            pltpu.SemaphoreType.DMA,
        ],
        compiler_params=pltpu.CompilerParams(needs_layout_passes=False),
    )
    def build(src_hbm, dst_hbm, f_hbm, self_hbm, sl_hbm, dl_hbm, cnt_hbm,
              s_v, d_v, sel_s, sel_d, acc, selfc, cnts, wb_sem):
        wid = lax.axis_index("s") * _NC + lax.axis_index("c")
        base = wid * w_words
        zero16 = jnp.zeros((16,), jnp.int32)
        lane0 = lax.iota(jnp.int32, 16) == 0

        def zero_acc(i, _):
            acc[pl.ds(i * 16, 16)] = zero16
            return 0

        lax.fori_loop(0, (w_words * w_words) // 16, zero_acc, 0)

        def zero_selfc(i, _):
            selfc[pl.ds(i * 16, 16)] = zero16
            return 0

        lax.fori_loop(0, w_words // 16 + 1, zero_selfc, 0)

        def do_chunk(c, _):
            pltpu.sync_copy(src_hbm.at[pl.ds(c * chunk, chunk)], s_v)
            pltpu.sync_copy(dst_hbm.at[pl.ds(c * chunk, chunk)], d_v)

            def compact(g, cnt):
                s16 = s_v[pl.ds(g * 16, 16)]
                d16 = d_v[pl.ds(g * 16, 16)]
                m = jnp.logical_and(s16 >= base, s16 < base + w_words)
                mi = jnp.where(m, 1, 0)
                cs = plsc.cumsum(mi)
                pos = cnt + cs - mi
                plsc.store_scatter(sel_s, [pos], s16 - base, mask=m)
                plsc.store_scatter(sel_d, [pos], d16, mask=m)
                return cnt + jnp.sum(mi)

            cnt = lax.fori_loop(0, groups, compact, 0)

            cwin = cnts[pl.ds(c, 16)]
            cnts[pl.ds(c, 16)] = jnp.where(lane0, cnt, cwin)
            pltpu.async_copy(sel_s.at[pl.ds(0, chunk)],
                             sl_hbm.at[pl.ds((wid * n_chunks + c) * chunk,
                                             chunk)], wb_sem)
            pltpu.async_copy(sel_d.at[pl.ds(0, chunk)],
                             dl_hbm.at[pl.ds((wid * n_chunks + c) * chunk,
                                             chunk)], wb_sem)

            def scatter_bit(p, _):
                s_loc = sel_s[pl.ds(p, 16)][0]
                dd = sel_d[pl.ds(p, 16)][0]
                wd = lax.rem(dd, w_words)
                bit = lax.div(dd, w_words)
                addr = s_loc * w_words + wd
                win = acc[pl.ds(addr, 16)]
                acc[pl.ds(addr, 16)] = jnp.where(
                    lane0, win | (jnp.int32(1) << bit), win)

                @pl.when(s_loc + base == dd)
                def _():
                    swin = selfc[pl.ds(s_loc, 16)]
                    selfc[pl.ds(s_loc, 16)] = jnp.where(lane0, swin + 1, swin)

                return 0

            lax.fori_loop(0, cnt, scatter_bit, 0)
            pltpu.make_async_copy(
                sel_s.at[pl.ds(0, chunk)],
                sl_hbm.at[pl.ds(wid * chunk, chunk)], wb_sem).wait()
            pltpu.make_async_copy(
                sel_d.at[pl.ds(0, chunk)],
                dl_hbm.at[pl.ds(wid * chunk, chunk)], wb_sem).wait()
            return 0

        lax.fori_loop(0, n_chunks, do_chunk, 0)

        pltpu.sync_copy(acc.at[pl.ds(0, w_words * w_words)],
                        f_hbm.at[pl.ds(base * w_words, w_words * w_words)])
        pltpu.sync_copy(selfc.at[pl.ds(0, w_words)],
                        self_hbm.at[pl.ds(base, w_words)])
        pltpu.sync_copy(cnts.at[pl.ds(0, nc_pad)],
                        cnt_hbm.at[pl.ds(wid * nc_pad, nc_pad)])

    return build(src, dst)


def _sc_two_hop(slist, dlist, counts, f2d, n_pad, w_words, e_pad, chunk):
    n_chunks = e_pad // chunk
    nc_pad = _round_up(n_chunks, 8)
    g_pitch = f2d.shape[1]

    mesh = plsc.VectorSubcoreMesh(core_axis_name="c", subcore_axis_name="s",
                                  num_cores=_NC, num_subcores=_NS)

    @functools.partial(
        pl.kernel,
        mesh=mesh,
        out_type=jax.ShapeDtypeStruct((n_pad * w_words,), jnp.int32),
        scratch_types=[
            pltpu.VMEM((chunk + 48,), jnp.int32),
            pltpu.VMEM((chunk + 48,), jnp.int32),
            pltpu.VMEM((nc_pad + 16,), jnp.int32),
            pltpu.VMEM(((w_words + 1) * w_words,), jnp.int32),
            pltpu.VMEM((16, g_pitch), jnp.int32),
            pltpu.VMEM((16, g_pitch), jnp.int32),
            pltpu.VMEM((16, g_pitch), jnp.int32),
            pltpu.SemaphoreType.DMA,
            pltpu.SemaphoreType.DMA,
            pltpu.SemaphoreType.DMA,
        ],
        compiler_params=pltpu.CompilerParams(needs_layout_passes=False),
    )
    def twohop(sl_hbm, dl_hbm, cnt_hbm, f_hbm, two_hbm, sel_s, sel_d, cnts,
               acc, r0, r1, r2, sm0, sm1, sm2):
        wid = lax.axis_index("s") * _NC + lax.axis_index("c")
        zero16 = jnp.zeros((16,), jnp.int32)
        bufs = ((r0, sm0), (r1, sm1), (r2, sm2))

        def zero_acc(i, _):
            acc[pl.ds(i * 16, 16)] = zero16
            return 0

        lax.fori_loop(0, (w_words * w_words) // 16, zero_acc, 0)

        pltpu.sync_copy(cnt_hbm.at[pl.ds(wid * nc_pad, nc_pad)],
                        cnts.at[pl.ds(0, nc_pad)])

        def issue(g, b):
            rbuf, sem = bufs[b]
            pltpu.async_copy(f_hbm.at[sel_d.at[pl.ds(g * 16, 16)]], rbuf, sem)

        def do_chunk(c, _):
            pltpu.sync_copy(
                sl_hbm.at[pl.ds((wid * n_chunks + c) * chunk, chunk)],
                sel_s.at[pl.ds(0, chunk)])
            pltpu.sync_copy(
                dl_hbm.at[pl.ds((wid * n_chunks + c) * chunk, chunk)],
                sel_d.at[pl.ds(0, chunk)])
            cnt = cnts[pl.ds(c, 16)][0]

            for t in range(3):
                sel_s[pl.ds(cnt + 16 * t, 16)] = jnp.full(
                    (16,), w_words, jnp.int32)
                sel_d[pl.ds(cnt + 16 * t, 16)] = zero16

            ngroups = lax.div(cnt + 15, 16)

            @pl.when(ngroups > 0)
            def _():
                issue(jnp.int32(0), 0)

            @pl.when(ngroups > 1)
            def _():
                issue(jnp.int32(1), 1)

            def or_group(g, b):
                rbuf, sem = bufs[b]
                pltpu.make_async_copy(
                    f_hbm.at[sel_d.at[pl.ds(0, 16)]], rbuf, sem).wait()

                @pl.when(g + 2 < ngroups)
                def _():
                    issue(g + 2, (b + 2) % 3)

                svec = sel_s[pl.ds(g * 16, 16)]
                for l in range(16):
                    s_loc = svec[l]
                    for cc in range(w_words // 16):
                        off = s_loc * w_words + cc * 16
                        acc[pl.ds(off, 16)] = (acc[pl.ds(off, 16)]
                                               | rbuf[l, pl.ds(cc * 16, 16)])

            def do_trip(t, _):
                g = t * 3

                @pl.when(g < ngroups)
                def _():
                    or_group(g, 0)

                @pl.when(g + 1 < ngroups)
                def _():
                    or_group(g + 1, 1)

                @pl.when(g + 2 < ngroups)
                def _():
                    or_group(g + 2, 2)

                return 0

            lax.fori_loop(0, (ngroups + 2) // 3, do_trip, 0)
            return 0

        lax.fori_loop(0, n_chunks, do_chunk, 0)

        pltpu.sync_copy(
            acc.at[pl.ds(0, w_words * w_words)],
            two_hbm.at[pl.ds(wid * w_words * w_words, w_words * w_words)])

    return twohop(slist, dlist, counts, f2d)


def _unpack_chunks(f_blk, a2w, bi, w_words):
    one = jnp.int32(1)
    for j in range(32):
        a1 = ((f_blk >> j) & one).astype(jnp.float32)
        a2 = ((a2w >> j) & one).astype(jnp.float32)
        yield j, a1, a2


def _diag_terms(f_blk, a2w, sc_blk, i0, bi, w_words):
    rows = i0 + lax.broadcasted_iota(jnp.int32, (bi, w_words), 0)
    cols = lax.broadcasted_iota(jnp.int32, (bi, w_words), 1)
    sel = cols == lax.rem(rows, w_words)
    fw = jnp.sum(jnp.where(sel, f_blk, 0), axis=1, keepdims=True)
    aw = jnp.sum(jnp.where(sel, a2w, 0), axis=1, keepdims=True)
    bitpos = lax.div(i0 + lax.broadcasted_iota(jnp.int32, (bi, 1), 0),
                     w_words)
    one = jnp.int32(1)
    d1 = ((fw >> bitpos) & one).astype(jnp.float32)
    d2 = ((aw >> bitpos) & one).astype(jnp.float32)
    self2 = (sc_blk >= 2).astype(jnp.float32)
    return self2 - d1, -d2


def _tc_degrees(f2d, two2d, selfcnt2d, r, n_pad, w_words, bi):
    nb = n_pad // bi
    hid = r.shape[1]

    def body(f_ref, t_ref, sc_ref, r_ref, d1_ref, d2_ref, rv1_ref, rv2_ref):
        i0 = pl.program_id(0) * bi
        f_blk = f_ref[...]
        a2w = t_ref[...] & ~f_blk
        d1 = jnp.zeros((bi, 1), jnp.float32)
        d2 = jnp.zeros((bi, 1), jnp.float32)
        for j, a1, a2 in _unpack_chunks(f_blk, a2w, bi, w_words):
            d1 = d1 + jnp.sum(a1, axis=1, keepdims=True)
            d2 = d2 + jnp.sum(a2, axis=1, keepdims=True)
        c1, c2 = _diag_terms(f_blk, a2w, sc_ref[...], i0, bi, w_words)
        d1 = d1 + c1
        d2 = d2 + c2
        d1 = jnp.where(d1 > 0, lax.rsqrt(d1), 0.0)
        d2 = jnp.where(d2 > 0, lax.rsqrt(d2), 0.0)
        d1_ref[...] = d1
        d2_ref[...] = d2
        rv1_ref[...] = r_ref[...] * d1
        rv2_ref[...] = r_ref[...] * d2

    return pl.pallas_call(
        body,
        grid=(nb,),
        in_specs=[
            pl.BlockSpec((bi, w_words), lambda i: (i, 0)),
            pl.BlockSpec((bi, w_words), lambda i: (i, 0)),
            pl.BlockSpec((bi, 1), lambda i: (i, 0)),
            pl.BlockSpec((bi, hid), lambda i: (i, 0)),
        ],
        out_specs=[
            pl.BlockSpec((bi, 1), lambda i: (i, 0)),
            pl.BlockSpec((bi, 1), lambda i: (i, 0)),
            pl.BlockSpec((bi, hid), lambda i: (i, 0)),
            pl.BlockSpec((bi, hid), lambda i: (i, 0)),
        ],
        out_shape=[
            jax.ShapeDtypeStruct((n_pad, 1), jnp.float32),
            jax.ShapeDtypeStruct((n_pad, 1), jnp.float32),
            jax.ShapeDtypeStruct((n_pad, hid), jnp.float32),
            jax.ShapeDtypeStruct((n_pad, hid), jnp.float32),
        ],
        compiler_params=pltpu.CompilerParams(
            dimension_semantics=("parallel",)),
    )(f2d, two2d, selfcnt2d, r)


def _tc_embed(x_pad, w_embed, n_pad, bi):
    nb = n_pad // bi
    feat = x_pad.shape[1]
    hid = w_embed.shape[1]

    def body(x_ref, w_ref, r_ref):
        r_ref[...] = jnp.maximum(
            jnp.dot(x_ref[...], w_ref[...],
                    preferred_element_type=jnp.float32), 0.0)

    return pl.pallas_call(
        body,
        grid=(nb,),
        in_specs=[
            pl.BlockSpec((bi, feat), lambda i: (i, 0)),
            pl.BlockSpec((feat, hid), lambda i: (0, 0)),
        ],
        out_specs=pl.BlockSpec((bi, hid), lambda i: (i, 0)),
        out_shape=jax.ShapeDtypeStruct((n_pad, hid), jnp.float32),
        compiler_params=pltpu.CompilerParams(
            dimension_semantics=("parallel",)),
    )(x_pad, w_embed)


def _tc_hop1(f2d, two2d, selfcnt2d, d1inv, d2inv, rv1, rv2, n_pad, w_words,
             bi):
    nb = n_pad // bi
    hid = rv1.shape[1]

    def body(f_ref, t_ref, sc_ref, d1_ref, d2_ref, v1_ref, v2_ref,
             y_ref, yv1_ref, yv2_ref):
        i0 = pl.program_id(0) * bi
        f_blk = f_ref[...]
        a2w = t_ref[...] & ~f_blk
        acc1 = jnp.zeros((bi, hid), jnp.float32)
        acc2 = jnp.zeros((bi, hid), jnp.float32)
        for j, a1, a2 in _unpack_chunks(f_blk, a2w, bi, w_words):
            v1 = v1_ref[pl.ds(j * w_words, w_words), :]
            v2 = v2_ref[pl.ds(j * w_words, w_words), :]
            acc1 = acc1 + jnp.dot(a1, v1, preferred_element_type=jnp.float32)
            acc2 = acc2 + jnp.dot(a2, v2, preferred_element_type=jnp.float32)
        c1, c2 = _diag_terms(f_blk, a2w, sc_ref[...], i0, bi, w_words)
        acc1 = acc1 + c1 * v1_ref[pl.ds(i0, bi), :]
        acc2 = acc2 + c2 * v2_ref[pl.ds(i0, bi), :]
        d1 = d1_ref[...]
        d2 = d2_ref[...]
        y = jnp.concatenate([acc1 * d1, acc2 * d2], axis=1)
        y_ref[...] = y
        yv1_ref[...] = y * d1
        yv2_ref[...] = y * d2

    return pl.pallas_call(
        body,
        grid=(nb,),
        in_specs=[
            pl.BlockSpec((bi, w_words), lambda i: (i, 0)),
            pl.BlockSpec((bi, w_words), lambda i: (i, 0)),
            pl.BlockSpec((bi, 1), lambda i: (i, 0)),
            pl.BlockSpec((bi, 1), lambda i: (i, 0)),
            pl.BlockSpec((bi, 1), lambda i: (i, 0)),
            pl.BlockSpec((n_pad, hid), lambda i: (0, 0)),
            pl.BlockSpec((n_pad, hid), lambda i: (0, 0)),
        ],
        out_specs=[
            pl.BlockSpec((bi, 2 * hid), lambda i: (i, 0)),
            pl.BlockSpec((bi, 2 * hid), lambda i: (i, 0)),
            pl.BlockSpec((bi, 2 * hid), lambda i: (i, 0)),
        ],
        out_shape=[
            jax.ShapeDtypeStruct((n_pad, 2 * hid), jnp.float32),
            jax.ShapeDtypeStruct((n_pad, 2 * hid), jnp.float32),
            jax.ShapeDtypeStruct((n_pad, 2 * hid), jnp.float32),
        ],
        compiler_params=pltpu.CompilerParams(
            dimension_semantics=("parallel",)),
    )(f2d, two2d, selfcnt2d, d1inv, d2inv, rv1, rv2)


def _tc_hop2_classify(f2d, two2d, selfcnt2d, d1inv, d2inv, yv1, yv2, r, y,
                      w_classify, n_pad, w_words, bi):
    nb = n_pad // bi
    c2 = yv1.shape[1]
    hid = r.shape[1]
    cls = w_classify.shape[1]

    def body(f_ref, t_ref, sc_ref, d1_ref, d2_ref, v1_ref, v2_ref, r_ref,
             y_ref, wc_ref, out_ref):
        i0 = pl.program_id(0) * bi
        f_blk = f_ref[...]
        a2w = t_ref[...] & ~f_blk
        acc1 = jnp.zeros((bi, c2), jnp.float32)
        acc2 = jnp.zeros((bi, c2), jnp.float32)
        for j, a1, a2 in _unpack_chunks(f_blk, a2w, bi, w_words):
            v1 = v1_ref[pl.ds(j * w_words, w_words), :]
            v2 = v2_ref[pl.ds(j * w_words, w_words), :]
            acc1 = acc1 + jnp.dot(a1, v1, preferred_element_type=jnp.float32)
            acc2 = acc2 + jnp.dot(a2, v2, preferred_element_type=jnp.float32)
        cc1, cc2 = _diag_terms(f_blk, a2w, sc_ref[...], i0, bi, w_words)
        acc1 = acc1 + cc1 * v1_ref[pl.ds(i0, bi), :]
        acc2 = acc2 + cc2 * v2_ref[pl.ds(i0, bi), :]
        z1 = acc1 * d1_ref[...]
        z2 = acc2 * d2_ref[...]
        out = jnp.dot(r_ref[...], wc_ref[pl.ds(0, hid), :],
                      preferred_element_type=jnp.float32)
        out = out + jnp.dot(y_ref[...], wc_ref[pl.ds(hid, c2), :],
                            preferred_element_type=jnp.float32)
        out = out + jnp.dot(z1, wc_ref[pl.ds(hid + c2, c2), :],
                            preferred_element_type=jnp.float32)
        out = out + jnp.dot(z2, wc_ref[pl.ds(hid + 2 * c2, c2), :],
                            preferred_element_type=jnp.float32)
        out_ref[...] = out

    return pl.pallas_call(
        body,
        grid=(nb,),
        in_specs=[
            pl.BlockSpec((bi, w_words), lambda i: (i, 0)),
            pl.BlockSpec((bi, w_words), lambda i: (i, 0)),
            pl.BlockSpec((bi, 1), lambda i: (i, 0)),
            pl.BlockSpec((bi, 1), lambda i: (i, 0)),
            pl.BlockSpec((bi, 1), lambda i: (i, 0)),
            pl.BlockSpec((n_pad, c2), lambda i: (0, 0)),
            pl.BlockSpec((n_pad, c2), lambda i: (0, 0)),
            pl.BlockSpec((bi, hid), lambda i: (i, 0)),
            pl.BlockSpec((bi, c2), lambda i: (i, 0)),
            pl.BlockSpec(w_classify.shape, lambda i: (0, 0)),
        ],
        out_specs=pl.BlockSpec((bi, cls), lambda i: (i, 0)),
        out_shape=jax.ShapeDtypeStruct((n_pad, cls), jnp.float32),
        compiler_params=pltpu.CompilerParams(
            dimension_semantics=("parallel",)),
    )(f2d, two2d, selfcnt2d, d1inv, d2inv, yv1, yv2, r, y, w_classify)


def _tc_pipeline(f2d, two2d, selfcnt2d, x, w_embed, w_classify, n, n_pad,
                 w_words, bi):
    x_pad = jnp.pad(x, ((0, n_pad - n), (0, 0)))
    r = _tc_embed(x_pad, w_embed, n_pad, bi)

    d1inv, d2inv, rv1, rv2 = _tc_degrees(f2d, two2d, selfcnt2d, r, n_pad,
                                         w_words, bi)

    y, yv1, yv2 = _tc_hop1(f2d, two2d, selfcnt2d, d1inv, d2inv, rv1, rv2,
                           n_pad, w_words, bi)

    out = _tc_hop2_classify(f2d, two2d, selfcnt2d, d1inv, d2inv, yv1, yv2,
                            r, y, w_classify, n_pad, w_words, bi)
    return out[:n]


def kernel(x, edge_index, w_embed, w_classify):
    n = x.shape[0]
    e = edge_index.shape[1]

    w_words = _round_up(-(-n // 32), 64)
    n_pad = 32 * w_words
    bi = 256

    chunk = 1600
    e_pad = _round_up(e, chunk)

    src = edge_index[0]
    dst = edge_index[1]
    if e_pad != e:
        src = jnp.concatenate(
            [src, jnp.full((e_pad - e,), jnp.int32(2 ** 30))])
        dst = jnp.concatenate([dst, jnp.zeros((e_pad - e,), jnp.int32)])

    f_flat, selfcnt, slist, dlist, counts = _sc_build_adj(
        src, dst, n_pad, w_words, e_pad, chunk)
    f2d = f_flat.reshape(n_pad, w_words)
    g_pitch = _round_up(w_words, 128)
    f2d_g = jnp.pad(f2d, ((0, 0), (0, g_pitch - w_words)))
    two_flat = _sc_two_hop(slist, dlist, counts, f2d_g, n_pad, w_words,
                           e_pad, chunk)
    two2d = two_flat.reshape(n_pad, w_words)
    selfcnt2d = selfcnt.reshape(n_pad, 1)

    return _tc_pipeline(f2d, two2d, selfcnt2d, x, w_embed, w_classify, n,
                        n_pad, w_words, bi)
